# baseline (speedup 1.0000x reference)
import functools

import jax
import jax.numpy as jnp
from jax import lax
from jax.experimental import pallas as pl
from jax.experimental.pallas import tpu as pltpu

_INTERPRET = False

K = 30
KP = 32
B, N = 16, 2048
TN = 256


def _fold_bn(W, b, gamma, beta):
    s = gamma / jnp.sqrt(1.0 + 1e-5)
    return W * s[None, :], b * s + beta


def _scores_kernel(x_ref, xt_ref, wu_ref, bu_ref, wv_ref, s_ref, u_ref, v_ref):
    xall = x_ref[0]
    xt = xt_ref[0]
    g = lax.dot_general(xt, xall, (((1,), (1,)), ((), ())),
                        preferred_element_type=jnp.float32)
    x2 = jnp.sum(xall * xall, axis=1)
    s_ref[0] = 2.0 * g - x2[None, :]
    u_ref[0] = jnp.dot(xt, wu_ref[...], preferred_element_type=jnp.float32) + bu_ref[...]
    v_ref[0] = jnp.dot(xt, wv_ref[...], preferred_element_type=jnp.float32)


def _scores(x, wu, bu, wv):
    d = x.shape[-1]
    grid = (B, N // TN)
    return pl.pallas_call(
        _scores_kernel,
        grid=grid,
        in_specs=[
            pl.BlockSpec((1, N, d), lambda b, t: (b, 0, 0)),
            pl.BlockSpec((1, TN, d), lambda b, t: (b, t, 0)),
            pl.BlockSpec((d, 32), lambda b, t: (0, 0)),
            pl.BlockSpec((1, 32), lambda b, t: (0, 0)),
            pl.BlockSpec((d, 32), lambda b, t: (0, 0)),
        ],
        out_specs=[
            pl.BlockSpec((1, TN, N), lambda b, t: (b, t, 0)),
            pl.BlockSpec((1, TN, 32), lambda b, t: (b, t, 0)),
            pl.BlockSpec((1, TN, 32), lambda b, t: (b, t, 0)),
        ],
        out_shape=[
            jax.ShapeDtypeStruct((B, N, N), jnp.float32),
            jax.ShapeDtypeStruct((B, N, 32), jnp.float32),
            jax.ShapeDtypeStruct((B, N, 32), jnp.float32),
        ],
        interpret=_INTERPRET,
    )(x, x, wu, bu, wv)


def _edge_kernel(u_ref, ve_ref, w2_ref, b2_ref, w3_ref, b3_ref, o_ref):
    u = u_ref[0]
    ve = ve_ref[0]
    ub = jnp.broadcast_to(u[:, None, :], (TN, KP, 32)).reshape(TN * KP, 32)
    h1 = jnp.maximum(ve + ub, 0.0)
    h2 = jnp.dot(h1, w2_ref[...], preferred_element_type=jnp.float32) + b2_ref[...]
    h2 = jnp.maximum(h2, 0.0)
    msg = jnp.dot(h2, w3_ref[...], preferred_element_type=jnp.float32) + b3_ref[...]
    o_ref[0] = jnp.max(msg.reshape(TN, KP, 32), axis=1)


def _edge_mlp_max(u, ve, w2, b2, w3, b3):
    grid = (B, N // TN)
    return pl.pallas_call(
        _edge_kernel,
        grid=grid,
        in_specs=[
            pl.BlockSpec((1, TN, 32), lambda b, t: (b, t, 0)),
            pl.BlockSpec((1, TN * KP, 32), lambda b, t: (b, t, 0)),
            pl.BlockSpec((32, 32), lambda b, t: (0, 0)),
            pl.BlockSpec((1, 32), lambda b, t: (0, 0)),
            pl.BlockSpec((32, 32), lambda b, t: (0, 0)),
            pl.BlockSpec((1, 32), lambda b, t: (0, 0)),
        ],
        out_specs=pl.BlockSpec((1, TN, 32), lambda b, t: (b, t, 0)),
        out_shape=jax.ShapeDtypeStruct((B, N, 32), jnp.float32),
        interpret=_INTERPRET,
    )(u, ve, w2, b2, w3, b3)


def _pool_kernel(x1_ref, x2_ref, x3_ref, w_ref, b_ref, o_ref):
    t = pl.program_id(1)
    h = jnp.concatenate([x1_ref[0], x2_ref[0], x3_ref[0]], axis=1)
    h = jnp.dot(h, w_ref[...], preferred_element_type=jnp.float32) + b_ref[...]
    m = jnp.max(h, axis=0, keepdims=True)[None]

    @pl.when(t == 0)
    def _():
        o_ref[...] = m

    @pl.when(t != 0)
    def _():
        o_ref[...] = jnp.maximum(o_ref[...], m)


def _pool(x1, x2, x3, w, b):
    grid = (B, N // TN)
    return pl.pallas_call(
        _pool_kernel,
        grid=grid,
        in_specs=[
            pl.BlockSpec((1, TN, 32), lambda b, t: (b, t, 0)),
            pl.BlockSpec((1, TN, 32), lambda b, t: (b, t, 0)),
            pl.BlockSpec((1, TN, 32), lambda b, t: (b, t, 0)),
            pl.BlockSpec((96, 1024), lambda b, t: (0, 0)),
            pl.BlockSpec((1, 1024), lambda b, t: (0, 0)),
        ],
        out_specs=pl.BlockSpec((1, 1, 1024), lambda b, t: (b, 0, 0)),
        out_shape=jax.ShapeDtypeStruct((B, 1, 1024), jnp.float32),
        interpret=_INTERPRET,
    )(x1, x2, x3, w, b).reshape(B, 1024)


def _head_kernel(g_ref, w0, b0, w1, b1, w2, b2, w3, b3, o_ref):
    g = g_ref[...]
    g = jnp.maximum(jnp.dot(g, w0[...], preferred_element_type=jnp.float32) + b0[...], 0.0)
    g = jnp.maximum(jnp.dot(g, w1[...], preferred_element_type=jnp.float32) + b1[...], 0.0)
    g = jnp.maximum(jnp.dot(g, w2[...], preferred_element_type=jnp.float32) + b2[...], 0.0)
    o_ref[...] = jnp.dot(g, w3[...], preferred_element_type=jnp.float32) + b3[...]


def _head(g, ws):
    ins = []
    specs = [pl.BlockSpec(g.shape, lambda: (0, 0))]
    for w, b in ws:
        ins += [w, b]
        specs += [pl.BlockSpec(w.shape, lambda: (0, 0)),
                  pl.BlockSpec(b.shape, lambda: (0, 0))]
    ncls = ws[-1][0].shape[1]
    return pl.pallas_call(
        _head_kernel,
        in_specs=specs,
        out_specs=pl.BlockSpec((B, ncls), lambda: (0, 0)),
        out_shape=jax.ShapeDtypeStruct((B, ncls), jnp.float32),
        interpret=_INTERPRET,
    )(g, *ins)


def _layer(x, layers):
    d = x.shape[-1]
    p1, p2, p3 = layers
    w1t, b1t = _fold_bn(p1['W'], p1['b'], p1['gamma'], p1['beta'])
    a, c = w1t[:d], w1t[d:]
    wu = a - c
    bu = b1t.reshape(1, 32)
    w2t, b2t = _fold_bn(p2['W'], p2['b'], p2['gamma'], p2['beta'])
    s, u, v = _scores(x, wu, bu, c)
    _, idx = lax.top_k(s, K)
    idxp = jnp.concatenate([idx, idx[..., :KP - K]], axis=-1)
    ve = jax.vmap(lambda vb, ib: vb[ib])(v, idxp)
    ve = ve.reshape(B, N * KP, 32)
    return _edge_mlp_max(u, ve, w2t, b2t.reshape(1, 32),
                         p3['W'], p3['b'].reshape(1, 32))


def kernel(data, params):
    x = data
    xs = []
    for li in range(3):
        x = _layer(x, params['conv%d' % li])
        xs.append(x)
    g = _pool(xs[0], xs[1], xs[2], params['lin1']['W'],
              params['lin1']['b'].reshape(1, 1024))
    ws = [(p['W'], p['b'].reshape(1, -1)) for p in params['out']]
    return _head(g, ws)


# speedup vs baseline: 15.1612x; 15.1612x over previous
import functools

import jax
import jax.numpy as jnp
from jax import lax
from jax.experimental import pallas as pl
from jax.experimental.pallas import tpu as pltpu
from jax.experimental.pallas import tpu_sc as plsc

_INTERPRET = False

K = 30
KP = 32
B, N = 16, 2048
TN = 256
GS = 16
G = N // GS
ROWS = B * N
NW = 32
RPW = ROWS // NW
CH = 64
ECH = 1024


def _fold_bn(W, b, gamma, beta):
    s = gamma / jnp.sqrt(1.0 + 1e-5)
    return W * s[None, :], b * s + beta


def _scores_kernel(x_ref, xt_ref, wu_ref, bu_ref, wv_ref, e_ref,
                   s_ref, u_ref, v_ref, m_ref):
    xall = x_ref[0]
    xt = xt_ref[0]
    g = lax.dot_general(xt, xall, (((1,), (1,)), ((), ())),
                        preferred_element_type=jnp.float32)
    x2 = jnp.sum(xall * xall, axis=1)
    s = 2.0 * g - x2[None, :]
    s_ref[0] = s
    t = s
    for k in (1, 2, 4, 8):
        pad = jnp.full((TN, k), -jnp.inf, jnp.float32)
        t = jnp.maximum(t, jnp.concatenate([t[:, k:], pad], axis=1))
    m_ref[0] = jnp.dot(t, e_ref[...], preferred_element_type=jnp.float32)
    u_ref[0] = jnp.dot(xt, wu_ref[...], preferred_element_type=jnp.float32) + bu_ref[...]
    v_ref[0] = jnp.dot(xt, wv_ref[...], preferred_element_type=jnp.float32)


def _scores(x, wu, bu, wv):
    d = x.shape[-1]
    grid = (B, N // TN)
    ext = jnp.zeros((N, G), jnp.float32).at[
        16 * jnp.arange(G), jnp.arange(G)].set(1.0)
    return pl.pallas_call(
        _scores_kernel,
        grid=grid,
        in_specs=[
            pl.BlockSpec((1, N, d), lambda b, t: (b, 0, 0)),
            pl.BlockSpec((1, TN, d), lambda b, t: (b, t, 0)),
            pl.BlockSpec((d, 32), lambda b, t: (0, 0)),
            pl.BlockSpec((1, 32), lambda b, t: (0, 0)),
            pl.BlockSpec((d, 32), lambda b, t: (0, 0)),
            pl.BlockSpec((N, G), lambda b, t: (0, 0)),
        ],
        out_specs=[
            pl.BlockSpec((1, TN, N), lambda b, t: (b, t, 0)),
            pl.BlockSpec((1, TN, 32), lambda b, t: (b, t, 0)),
            pl.BlockSpec((1, TN, 32), lambda b, t: (b, t, 0)),
            pl.BlockSpec((1, TN, G), lambda b, t: (b, t, 0)),
        ],
        out_shape=[
            jax.ShapeDtypeStruct((B, N, N), jnp.float32),
            jax.ShapeDtypeStruct((B, N, 32), jnp.float32),
            jax.ShapeDtypeStruct((B, N, 32), jnp.float32),
            jax.ShapeDtypeStruct((B, N, G), jnp.float32),
        ],
        interpret=_INTERPRET,
    )(x, x, wu, bu, wv, ext)


def _sort16(k, v):
    return plsc.sort_key_val(k, v, descending=True)


def _merge16kv(ak, av, bk, bv):
    brk, brv = lax.rev(bk, (0,)), lax.rev(bv, (0,))
    m = ak >= brk
    hk = jnp.where(m, ak, brk)
    hv = jnp.where(m, av, brv)
    lk = jnp.where(m, brk, ak)
    lv = jnp.where(m, brv, av)
    hk, hv = _sort16(hk, hv)
    lk, lv = _sort16(lk, lv)
    return hk, hv, lk, lv


def _merge32kv(a, b):
    a0k, a0v, a1k, a1v = a
    b0k, b0v, b1k, b1v = b
    r1k, r1v = lax.rev(b1k, (0,)), lax.rev(b1v, (0,))
    r0k, r0v = lax.rev(b0k, (0,)), lax.rev(b0v, (0,))
    m0 = a0k >= r1k
    l0k = jnp.where(m0, a0k, r1k)
    l0v = jnp.where(m0, a0v, r1v)
    m1 = a1k >= r0k
    l1k = jnp.where(m1, a1k, r0k)
    l1v = jnp.where(m1, a1v, r0v)
    m2 = l0k >= l1k
    hk = jnp.where(m2, l0k, l1k)
    hv = jnp.where(m2, l0v, l1v)
    lk = jnp.where(m2, l1k, l0k)
    lv = jnp.where(m2, l1v, l0v)
    hk, hv = _sort16(hk, hv)
    lk, lv = _sort16(lk, lv)
    return hk, hv, lk, lv


def _top32_net(pairs):
    units = [_merge16kv(*pairs[2 * j], *pairs[2 * j + 1])
             for j in range(len(pairs) // 2)]
    while len(units) > 1:
        units = [_merge32kv(units[2 * j], units[2 * j + 1])
                 for j in range(len(units) // 2)]
    return units[0]


SCH = 16
NCH = RPW // SCH


def _topk_sc(s2, m1):
    mesh = plsc.VectorSubcoreMesh(core_axis_name="c", subcore_axis_name="s")

    @functools.partial(
        pl.kernel,
        mesh=mesh,
        out_type=jax.ShapeDtypeStruct((ROWS * KP,), jnp.int32),
        compiler_params=pltpu.CompilerParams(needs_layout_passes=False),
        interpret=_INTERPRET,
        scratch_types=[
            pltpu.VMEM((2, SCH, N), jnp.float32),
            pltpu.VMEM((2, SCH, G), jnp.float32),
            pltpu.VMEM((SCH * KP,), jnp.int32),
            pltpu.VMEM((KP,), jnp.int32),
            pltpu.SemaphoreType.DMA,
            pltpu.SemaphoreType.DMA,
            pltpu.SemaphoreType.DMA,
            pltpu.SemaphoreType.DMA,
        ],
    )
    def topk_kernel(s_hbm, m_hbm, o_hbm, s_buf, m_buf, out_buf, idxv,
                    ss0, ss1, ms0, ms1):
        wid = lax.axis_index("s") * 2 + lax.axis_index("c")
        wbase = wid * RPW
        iot = lax.broadcasted_iota(jnp.int32, (GS,), 0)
        ssem = (ss0, ss1)
        msem = (ms0, ms1)

        def start(c, par):
            rb = pl.multiple_of(wbase + c * SCH, SCH)
            pltpu.make_async_copy(
                s_hbm.at[pl.ds(rb, SCH)], s_buf.at[par], ssem[par]).start()
            pltpu.make_async_copy(
                m_hbm.at[pl.ds(rb, SCH)], m_buf.at[par], msem[par]).start()

        def wait(par):
            pltpu.make_async_copy(
                s_hbm.at[pl.ds(0, SCH)], s_buf.at[par], ssem[par]).wait()
            pltpu.make_async_copy(
                m_hbm.at[pl.ds(0, SCH)], m_buf.at[par], msem[par]).wait()

        def compute(c, par):
            rbase = wbase + c * SCH

            def row_body(r, carry_r):
                rabs = rbase + r
                rsplat = jnp.full((GS,), r, jnp.int32)
                leaves = []
                for j in range(G // GS):
                    kj = plsc.load_gather(
                        m_buf.at[par], [rsplat, iot + (GS * j)])
                    leaves.append(_sort16(kj, iot + (GS * j)))
                _, ghv, _, glv = _top32_net(leaves)
                idxv[pl.ds(0, GS)] = ghv
                idxv[pl.ds(GS, GS)] = glv
                el = []
                for j in range(KP):
                    g = plsc.load_gather(idxv, [jnp.full((GS,), j, jnp.int32)])
                    col = g * GS + iot
                    kj = plsc.load_gather(s_buf.at[par], [rsplat, col])
                    el.append(_sort16(kj, col))
                _, hv, _, lv = _top32_net(el)
                oo = r * KP
                out_buf[pl.ds(oo, GS)] = hv
                pad = jnp.full((GS,), lax.rem(rabs, N), jnp.int32)
                lv = jnp.where(iot >= GS - 2, pad, lv)
                out_buf[pl.ds(oo + GS, GS)] = lv
                return carry_r

            lax.fori_loop(0, SCH, row_body, 0)
            pltpu.sync_copy(
                out_buf,
                o_hbm.at[pl.ds(pl.multiple_of(rbase * KP, SCH * KP), SCH * KP)])

        start(0, 0)

        def pair_body(t, carry):
            for par in range(2):
                c = 2 * t + par
                wait(par)

                @pl.when(c + 1 < NCH)
                def _():
                    start(c + 1, 1 - par)

                compute(c, par)
            return carry

        lax.fori_loop(0, NCH // 2, pair_body, 0)

    return topk_kernel(s2, m1)


def _gather_sc(v2, idx):
    mesh = plsc.VectorSubcoreMesh(core_axis_name="c", subcore_axis_name="s")
    epw = ROWS * KP // NW

    @functools.partial(
        pl.kernel,
        mesh=mesh,
        out_type=jax.ShapeDtypeStruct((ROWS * KP // 4, 128), jnp.float32),
        compiler_params=pltpu.CompilerParams(needs_layout_passes=False),
        interpret=_INTERPRET,
        scratch_types=[
            pltpu.VMEM((N * 32,), jnp.float32),
            pltpu.VMEM((ECH,), jnp.int32),
            pltpu.VMEM((ECH // 4, 128), jnp.float32),
        ],
    )
    def gather_kernel(v_hbm, i_hbm, o_hbm, vtab, ibuf, obuf):
        wid = lax.axis_index("s") * 2 + lax.axis_index("c")
        ebase = wid * epw
        b = wid // 2
        pltpu.sync_copy(
            v_hbm.at[pl.ds(pl.multiple_of(b * N * 32, N * 32), N * 32)], vtab)
        iot = lax.broadcasted_iota(jnp.int32, (GS,), 0)
        ec0 = (iot % 4) * 32
        iot4 = iot // 4

        def body(t, carry):
            off = ebase + t * ECH
            pltpu.sync_copy(
                i_hbm.at[pl.ds(pl.multiple_of(off, ECH), ECH)], ibuf)

            def edges16(e0, carry2):
                nids = ibuf[pl.ds(e0, GS)] * 32
                erow = (e0 // 4) + iot4
                for c in range(32):
                    vals = plsc.load_gather(vtab, [nids + c])
                    plsc.store_scatter(obuf, [erow, ec0 + c], vals)
                return carry2

            lax.fori_loop(0, ECH // GS, lambda i, c: edges16(i * GS, c), 0)
            pltpu.sync_copy(
                obuf, o_hbm.at[pl.ds(pl.multiple_of(off // 4, 256), ECH // 4)])
            return carry

        lax.fori_loop(0, epw // ECH, body, 0)

    return gather_kernel(v2, idx)


RP = TN * KP // 4


def _edge_kernel(u_ref, ve_ref, w2_ref, b2_ref, w3_ref, b3_ref, o_ref):
    u = u_ref[0]
    ve = ve_ref[0]
    u4 = jnp.tile(u, (1, 4))
    ub = jnp.broadcast_to(u4[:, None, :], (TN, KP // 4, 128)).reshape(RP, 128)
    h1 = jnp.maximum(ve + ub, 0.0)
    h2 = jnp.dot(h1, w2_ref[...], preferred_element_type=jnp.float32) + b2_ref[...]
    h2 = jnp.maximum(h2, 0.0)
    msg = jnp.dot(h2, w3_ref[...], preferred_element_type=jnp.float32) + b3_ref[...]
    t = jnp.max(msg.reshape(TN, KP // 4, 128), axis=1)
    o_ref[0] = jnp.maximum(
        jnp.maximum(t[:, 0:32], t[:, 32:64]),
        jnp.maximum(t[:, 64:96], t[:, 96:128]))


def _edge_mlp_max(u, ve, w2, b2, w3, b3):
    w2d = jnp.kron(jnp.eye(4, dtype=jnp.float32), w2)
    w3d = jnp.kron(jnp.eye(4, dtype=jnp.float32), w3)
    b2d = jnp.tile(b2, (1, 4))
    b3d = jnp.tile(b3, (1, 4))
    grid = (B, N // TN)
    return pl.pallas_call(
        _edge_kernel,
        grid=grid,
        in_specs=[
            pl.BlockSpec((1, TN, 32), lambda b, t: (b, t, 0)),
            pl.BlockSpec((1, RP, 128), lambda b, t: (b, t, 0)),
            pl.BlockSpec((128, 128), lambda b, t: (0, 0)),
            pl.BlockSpec((1, 128), lambda b, t: (0, 0)),
            pl.BlockSpec((128, 128), lambda b, t: (0, 0)),
            pl.BlockSpec((1, 128), lambda b, t: (0, 0)),
        ],
        out_specs=pl.BlockSpec((1, TN, 32), lambda b, t: (b, t, 0)),
        out_shape=jax.ShapeDtypeStruct((B, N, 32), jnp.float32),
        interpret=_INTERPRET,
    )(u, ve, w2d, b2d, w3d, b3d)


def _pool_kernel(x1_ref, x2_ref, x3_ref, w_ref, b_ref, o_ref):
    t = pl.program_id(1)
    h = jnp.concatenate([x1_ref[0], x2_ref[0], x3_ref[0]], axis=1)
    h = jnp.dot(h, w_ref[...], preferred_element_type=jnp.float32) + b_ref[...]
    m = jnp.max(h, axis=0, keepdims=True)[None]

    @pl.when(t == 0)
    def _():
        o_ref[...] = m

    @pl.when(t != 0)
    def _():
        o_ref[...] = jnp.maximum(o_ref[...], m)


def _pool(x1, x2, x3, w, b):
    grid = (B, N // TN)
    return pl.pallas_call(
        _pool_kernel,
        grid=grid,
        in_specs=[
            pl.BlockSpec((1, TN, 32), lambda b, t: (b, t, 0)),
            pl.BlockSpec((1, TN, 32), lambda b, t: (b, t, 0)),
            pl.BlockSpec((1, TN, 32), lambda b, t: (b, t, 0)),
            pl.BlockSpec((96, 1024), lambda b, t: (0, 0)),
            pl.BlockSpec((1, 1024), lambda b, t: (0, 0)),
        ],
        out_specs=pl.BlockSpec((1, 1, 1024), lambda b, t: (b, 0, 0)),
        out_shape=jax.ShapeDtypeStruct((B, 1, 1024), jnp.float32),
        interpret=_INTERPRET,
    )(x1, x2, x3, w, b).reshape(B, 1024)


def _head_kernel(g_ref, w0, b0, w1, b1, w2, b2, w3, b3, o_ref):
    g = g_ref[...]
    g = jnp.maximum(jnp.dot(g, w0[...], preferred_element_type=jnp.float32) + b0[...], 0.0)
    g = jnp.maximum(jnp.dot(g, w1[...], preferred_element_type=jnp.float32) + b1[...], 0.0)
    g = jnp.maximum(jnp.dot(g, w2[...], preferred_element_type=jnp.float32) + b2[...], 0.0)
    o_ref[...] = jnp.dot(g, w3[...], preferred_element_type=jnp.float32) + b3[...]


def _head(g, ws):
    ins = []
    specs = [pl.BlockSpec(g.shape, lambda: (0, 0))]
    for w, b in ws:
        ins += [w, b]
        specs += [pl.BlockSpec(w.shape, lambda: (0, 0)),
                  pl.BlockSpec(b.shape, lambda: (0, 0))]
    ncls = ws[-1][0].shape[1]
    return pl.pallas_call(
        _head_kernel,
        in_specs=specs,
        out_specs=pl.BlockSpec((B, ncls), lambda: (0, 0)),
        out_shape=jax.ShapeDtypeStruct((B, ncls), jnp.float32),
        interpret=_INTERPRET,
    )(g, *ins)


def _layer(x, layers):
    d = x.shape[-1]
    p1, p2, p3 = layers
    w1t, b1t = _fold_bn(p1['W'], p1['b'], p1['gamma'], p1['beta'])
    a, c = w1t[:d], w1t[d:]
    wu = a - c
    bu = b1t.reshape(1, 32)
    w2t, b2t = _fold_bn(p2['W'], p2['b'], p2['gamma'], p2['beta'])
    s, u, v, m = _scores(x, wu, bu, c)
    idxg = _topk_sc(s.reshape(ROWS, N), m.reshape(ROWS, G))
    ve = _gather_sc(v.reshape(B * N * 32), idxg)
    ve = ve.reshape(B, N * KP // 4, 128)
    return _edge_mlp_max(u, ve, w2t, b2t.reshape(1, 32),
                         p3['W'], p3['b'].reshape(1, 32))


def kernel(data, params):
    x = data
    xs = []
    for li in range(3):
        x = _layer(x, params['conv%d' % li])
        xs.append(x)
    g = _pool(xs[0], xs[1], xs[2], params['lin1']['W'],
              params['lin1']['b'].reshape(1, 1024))
    ws = [(p['W'], p['b'].reshape(1, -1)) for p in params['out']]
    return _head(g, ws)


# speedup vs baseline: 20.0394x; 1.3218x over previous
import functools

import jax
import jax.numpy as jnp
from jax import lax
from jax.experimental import pallas as pl
from jax.experimental.pallas import tpu as pltpu
from jax.experimental.pallas import tpu_sc as plsc

_INTERPRET = False

K = 30
KP = 32
B, N = 16, 2048
TN = 256
GS = 16
G = N // GS
ROWS = B * N
NW = 32
RPW = ROWS // NW
CH = 64
ECH = 1024
VP = 33


def _fold_bn(W, b, gamma, beta):
    s = gamma / jnp.sqrt(1.0 + 1e-5)
    return W * s[None, :], b * s + beta


def _scores_kernel(x_ref, xt_ref, wu_ref, bu_ref, wv_ref, e_ref,
                   s_ref, u_ref, v_ref, m_ref):
    xall = x_ref[0]
    xt = xt_ref[0]
    g = lax.dot_general(xt, xall, (((1,), (1,)), ((), ())),
                        preferred_element_type=jnp.float32)
    x2 = jnp.sum(xall * xall, axis=1)
    s = 2.0 * g - x2[None, :]
    s_ref[0] = s
    t = s
    for k in (1, 2, 4, 8):
        pad = jnp.full((TN, k), -jnp.inf, jnp.float32)
        t = jnp.maximum(t, jnp.concatenate([t[:, k:], pad], axis=1))
    m_ref[0] = jnp.dot(t, e_ref[...], preferred_element_type=jnp.float32)
    u_ref[0] = jnp.dot(xt, wu_ref[...], preferred_element_type=jnp.float32) + bu_ref[...]
    v_ref[0] = jnp.dot(xt, wv_ref[...], preferred_element_type=jnp.float32)


def _scores(x, wu, bu, wv):
    d = x.shape[-1]
    grid = (B, N // TN)
    ext = jnp.zeros((N, G), jnp.float32).at[
        16 * jnp.arange(G), jnp.arange(G)].set(1.0)
    return pl.pallas_call(
        _scores_kernel,
        grid=grid,
        in_specs=[
            pl.BlockSpec((1, N, d), lambda b, t: (b, 0, 0)),
            pl.BlockSpec((1, TN, d), lambda b, t: (b, t, 0)),
            pl.BlockSpec((d, 32), lambda b, t: (0, 0)),
            pl.BlockSpec((1, 32), lambda b, t: (0, 0)),
            pl.BlockSpec((d, 32), lambda b, t: (0, 0)),
            pl.BlockSpec((N, G), lambda b, t: (0, 0)),
        ],
        out_specs=[
            pl.BlockSpec((1, TN, N), lambda b, t: (b, t, 0)),
            pl.BlockSpec((1, TN, 32), lambda b, t: (b, t, 0)),
            pl.BlockSpec((1, TN, 32), lambda b, t: (b, t, 0)),
            pl.BlockSpec((1, TN, G), lambda b, t: (b, t, 0)),
        ],
        out_shape=[
            jax.ShapeDtypeStruct((B, N, N), jnp.float32),
            jax.ShapeDtypeStruct((B, N, 32), jnp.float32),
            jax.ShapeDtypeStruct((B, N, 32), jnp.float32),
            jax.ShapeDtypeStruct((B, N, G), jnp.float32),
        ],
        interpret=_INTERPRET,
    )(x, x, wu, bu, wv, ext)


def _sort16(k, v):
    return plsc.sort_key_val(k, v, descending=True)


def _merge16kv(ak, av, bk, bv):
    brk, brv = lax.rev(bk, (0,)), lax.rev(bv, (0,))
    m = ak >= brk
    hk = jnp.where(m, ak, brk)
    hv = jnp.where(m, av, brv)
    lk = jnp.where(m, brk, ak)
    lv = jnp.where(m, brv, av)
    hk, hv = _sort16(hk, hv)
    lk, lv = _sort16(lk, lv)
    return hk, hv, lk, lv


def _merge32kv(a, b):
    a0k, a0v, a1k, a1v = a
    b0k, b0v, b1k, b1v = b
    r1k, r1v = lax.rev(b1k, (0,)), lax.rev(b1v, (0,))
    r0k, r0v = lax.rev(b0k, (0,)), lax.rev(b0v, (0,))
    m0 = a0k >= r1k
    l0k = jnp.where(m0, a0k, r1k)
    l0v = jnp.where(m0, a0v, r1v)
    m1 = a1k >= r0k
    l1k = jnp.where(m1, a1k, r0k)
    l1v = jnp.where(m1, a1v, r0v)
    m2 = l0k >= l1k
    hk = jnp.where(m2, l0k, l1k)
    hv = jnp.where(m2, l0v, l1v)
    lk = jnp.where(m2, l1k, l0k)
    lv = jnp.where(m2, l1v, l0v)
    hk, hv = _sort16(hk, hv)
    lk, lv = _sort16(lk, lv)
    return hk, hv, lk, lv


def _top32_net(pairs):
    units = [_merge16kv(*pairs[2 * j], *pairs[2 * j + 1])
             for j in range(len(pairs) // 2)]
    while len(units) > 1:
        units = [_merge32kv(units[2 * j], units[2 * j + 1])
                 for j in range(len(units) // 2)]
    return units[0]


SCH = 16
NCH = RPW // SCH


def _topk_sc(s2, m1):
    mesh = plsc.VectorSubcoreMesh(core_axis_name="c", subcore_axis_name="s")

    @functools.partial(
        pl.kernel,
        mesh=mesh,
        out_type=jax.ShapeDtypeStruct((ROWS * KP,), jnp.int32),
        compiler_params=pltpu.CompilerParams(needs_layout_passes=False),
        interpret=_INTERPRET,
        scratch_types=[
            pltpu.VMEM((2, SCH, N), jnp.float32),
            pltpu.VMEM((2, SCH, G), jnp.float32),
            pltpu.VMEM((SCH * KP,), jnp.int32),
            pltpu.VMEM((KP,), jnp.int32),
            pltpu.SemaphoreType.DMA,
            pltpu.SemaphoreType.DMA,
            pltpu.SemaphoreType.DMA,
            pltpu.SemaphoreType.DMA,
        ],
    )
    def topk_kernel(s_hbm, m_hbm, o_hbm, s_buf, m_buf, out_buf, idxv,
                    ss0, ss1, ms0, ms1):
        wid = lax.axis_index("s") * 2 + lax.axis_index("c")
        wbase = wid * RPW
        iot = lax.broadcasted_iota(jnp.int32, (GS,), 0)
        ssem = (ss0, ss1)
        msem = (ms0, ms1)

        def start(c, par):
            rb = pl.multiple_of(wbase + c * SCH, SCH)
            pltpu.make_async_copy(
                s_hbm.at[pl.ds(rb, SCH)], s_buf.at[par], ssem[par]).start()
            pltpu.make_async_copy(
                m_hbm.at[pl.ds(rb, SCH)], m_buf.at[par], msem[par]).start()

        def wait(par):
            pltpu.make_async_copy(
                s_hbm.at[pl.ds(0, SCH)], s_buf.at[par], ssem[par]).wait()
            pltpu.make_async_copy(
                m_hbm.at[pl.ds(0, SCH)], m_buf.at[par], msem[par]).wait()

        def compute(c, par):
            rbase = wbase + c * SCH

            def row_body(r, carry_r):
                rabs = rbase + r
                rsplat = jnp.full((GS,), r, jnp.int32)
                leaves = []
                for j in range(G // GS):
                    kj = plsc.load_gather(
                        m_buf.at[par], [rsplat, iot + (GS * j)])
                    leaves.append(_sort16(kj, iot + (GS * j)))
                _, ghv, _, glv = _top32_net(leaves)
                idxv[pl.ds(0, GS)] = ghv
                idxv[pl.ds(GS, GS)] = glv
                el = []
                for j in range(KP):
                    g = plsc.load_gather(idxv, [jnp.full((GS,), j, jnp.int32)])
                    col = g * GS + iot
                    kj = plsc.load_gather(s_buf.at[par], [rsplat, col])
                    el.append(_sort16(kj, col))
                _, hv, _, lv = _top32_net(el)
                oo = r * KP
                out_buf[pl.ds(oo, GS)] = hv
                pad = jnp.full((GS,), lax.rem(rabs, N), jnp.int32)
                lv = jnp.where(iot >= GS - 2, pad, lv)
                out_buf[pl.ds(oo + GS, GS)] = lv
                return carry_r

            def row_pair(i, carry_r):
                row_body(2 * i, carry_r)
                row_body(2 * i + 1, carry_r)
                return carry_r

            lax.fori_loop(0, SCH // 2, row_pair, 0)
            pltpu.sync_copy(
                out_buf,
                o_hbm.at[pl.ds(pl.multiple_of(rbase * KP, SCH * KP), SCH * KP)])

        start(0, 0)

        def pair_body(t, carry):
            for par in range(2):
                c = 2 * t + par
                wait(par)

                @pl.when(c + 1 < NCH)
                def _():
                    start(c + 1, 1 - par)

                compute(c, par)
            return carry

        lax.fori_loop(0, NCH // 2, pair_body, 0)

    return topk_kernel(s2, m1)


def _gather_sc(v2, idx):
    mesh = plsc.VectorSubcoreMesh(core_axis_name="c", subcore_axis_name="s")
    epw = ROWS * KP // NW

    @functools.partial(
        pl.kernel,
        mesh=mesh,
        out_type=jax.ShapeDtypeStruct((ROWS * KP // 4, 128), jnp.float32),
        compiler_params=pltpu.CompilerParams(needs_layout_passes=False),
        interpret=_INTERPRET,
        scratch_types=[
            pltpu.VMEM((N * VP,), jnp.float32),
            pltpu.VMEM((ECH,), jnp.int32),
            pltpu.VMEM((ECH // 4, 128), jnp.float32),
        ],
    )
    def gather_kernel(v_hbm, i_hbm, o_hbm, vtab, ibuf, obuf):
        wid = lax.axis_index("s") * 2 + lax.axis_index("c")
        ebase = wid * epw
        b = wid // 2
        pltpu.sync_copy(
            v_hbm.at[pl.ds(pl.multiple_of(b * N * VP, N * VP), N * VP)], vtab)
        iot = lax.broadcasted_iota(jnp.int32, (GS,), 0)
        ec0 = (iot % 4) * 32
        iot4 = iot // 4

        def body(t, carry):
            off = ebase + t * ECH
            pltpu.sync_copy(
                i_hbm.at[pl.ds(pl.multiple_of(off, ECH), ECH)], ibuf)

            def edges16(e0, carry2):
                nids = ibuf[pl.ds(e0, GS)] * VP
                erow = (e0 // 4) + iot4
                for c in range(32):
                    vals = plsc.load_gather(vtab, [nids + c])
                    plsc.store_scatter(obuf, [erow, ec0 + c], vals)
                return carry2

            lax.fori_loop(0, ECH // GS, lambda i, c: edges16(i * GS, c), 0)
            pltpu.sync_copy(
                obuf, o_hbm.at[pl.ds(pl.multiple_of(off // 4, 256), ECH // 4)])
            return carry

        lax.fori_loop(0, epw // ECH, body, 0)

    return gather_kernel(v2, idx)


RP = TN * KP // 4


def _edge_kernel(u_ref, ve_ref, w2_ref, b2_ref, w3_ref, b3_ref, o_ref):
    u = u_ref[0]
    ve = ve_ref[0]
    u4 = jnp.tile(u, (1, 4))
    ub = jnp.broadcast_to(u4[:, None, :], (TN, KP // 4, 128)).reshape(RP, 128)
    h1 = jnp.maximum(ve + ub, 0.0)
    h2 = jnp.dot(h1, w2_ref[...], preferred_element_type=jnp.float32) + b2_ref[...]
    h2 = jnp.maximum(h2, 0.0)
    msg = jnp.dot(h2, w3_ref[...], preferred_element_type=jnp.float32) + b3_ref[...]
    t = jnp.max(msg.reshape(TN, KP // 4, 128), axis=1)
    o_ref[0] = jnp.maximum(
        jnp.maximum(t[:, 0:32], t[:, 32:64]),
        jnp.maximum(t[:, 64:96], t[:, 96:128]))


def _edge_mlp_max(u, ve, w2, b2, w3, b3):
    w2d = jnp.kron(jnp.eye(4, dtype=jnp.float32), w2)
    w3d = jnp.kron(jnp.eye(4, dtype=jnp.float32), w3)
    b2d = jnp.tile(b2, (1, 4))
    b3d = jnp.tile(b3, (1, 4))
    grid = (B, N // TN)
    return pl.pallas_call(
        _edge_kernel,
        grid=grid,
        in_specs=[
            pl.BlockSpec((1, TN, 32), lambda b, t: (b, t, 0)),
            pl.BlockSpec((1, RP, 128), lambda b, t: (b, t, 0)),
            pl.BlockSpec((128, 128), lambda b, t: (0, 0)),
            pl.BlockSpec((1, 128), lambda b, t: (0, 0)),
            pl.BlockSpec((128, 128), lambda b, t: (0, 0)),
            pl.BlockSpec((1, 128), lambda b, t: (0, 0)),
        ],
        out_specs=pl.BlockSpec((1, TN, 32), lambda b, t: (b, t, 0)),
        out_shape=jax.ShapeDtypeStruct((B, N, 32), jnp.float32),
        interpret=_INTERPRET,
    )(u, ve, w2d, b2d, w3d, b3d)


def _pool_kernel(x1_ref, x2_ref, x3_ref, w_ref, b_ref, o_ref):
    t = pl.program_id(1)
    h = jnp.concatenate([x1_ref[0], x2_ref[0], x3_ref[0]], axis=1)
    h = jnp.dot(h, w_ref[...], preferred_element_type=jnp.float32) + b_ref[...]
    m = jnp.max(h, axis=0, keepdims=True)[None]

    @pl.when(t == 0)
    def _():
        o_ref[...] = m

    @pl.when(t != 0)
    def _():
        o_ref[...] = jnp.maximum(o_ref[...], m)


def _pool(x1, x2, x3, w, b):
    grid = (B, N // TN)
    return pl.pallas_call(
        _pool_kernel,
        grid=grid,
        in_specs=[
            pl.BlockSpec((1, TN, 32), lambda b, t: (b, t, 0)),
            pl.BlockSpec((1, TN, 32), lambda b, t: (b, t, 0)),
            pl.BlockSpec((1, TN, 32), lambda b, t: (b, t, 0)),
            pl.BlockSpec((96, 1024), lambda b, t: (0, 0)),
            pl.BlockSpec((1, 1024), lambda b, t: (0, 0)),
        ],
        out_specs=pl.BlockSpec((1, 1, 1024), lambda b, t: (b, 0, 0)),
        out_shape=jax.ShapeDtypeStruct((B, 1, 1024), jnp.float32),
        interpret=_INTERPRET,
    )(x1, x2, x3, w, b).reshape(B, 1024)


def _head_kernel(g_ref, w0, b0, w1, b1, w2, b2, w3, b3, o_ref):
    g = g_ref[...]
    g = jnp.maximum(jnp.dot(g, w0[...], preferred_element_type=jnp.float32) + b0[...], 0.0)
    g = jnp.maximum(jnp.dot(g, w1[...], preferred_element_type=jnp.float32) + b1[...], 0.0)
    g = jnp.maximum(jnp.dot(g, w2[...], preferred_element_type=jnp.float32) + b2[...], 0.0)
    o_ref[...] = jnp.dot(g, w3[...], preferred_element_type=jnp.float32) + b3[...]


def _head(g, ws):
    ins = []
    specs = [pl.BlockSpec(g.shape, lambda: (0, 0))]
    for w, b in ws:
        ins += [w, b]
        specs += [pl.BlockSpec(w.shape, lambda: (0, 0)),
                  pl.BlockSpec(b.shape, lambda: (0, 0))]
    ncls = ws[-1][0].shape[1]
    return pl.pallas_call(
        _head_kernel,
        in_specs=specs,
        out_specs=pl.BlockSpec((B, ncls), lambda: (0, 0)),
        out_shape=jax.ShapeDtypeStruct((B, ncls), jnp.float32),
        interpret=_INTERPRET,
    )(g, *ins)


def _layer(x, layers):
    d = x.shape[-1]
    p1, p2, p3 = layers
    w1t, b1t = _fold_bn(p1['W'], p1['b'], p1['gamma'], p1['beta'])
    a, c = w1t[:d], w1t[d:]
    wu = a - c
    bu = b1t.reshape(1, 32)
    w2t, b2t = _fold_bn(p2['W'], p2['b'], p2['gamma'], p2['beta'])
    s, u, v, m = _scores(x, wu, bu, c)
    idxg = _topk_sc(s.reshape(ROWS, N), m.reshape(ROWS, G))
    vp = jnp.pad(v, ((0, 0), (0, 0), (0, VP - 32)))
    ve = _gather_sc(vp.reshape(B * N * VP), idxg)
    ve = ve.reshape(B, N * KP // 4, 128)
    return _edge_mlp_max(u, ve, w2t, b2t.reshape(1, 32),
                         p3['W'], p3['b'].reshape(1, 32))


def kernel(data, params):
    x = data
    xs = []
    for li in range(3):
        x = _layer(x, params['conv%d' % li])
        xs.append(x)
    g = _pool(xs[0], xs[1], xs[2], params['lin1']['W'],
              params['lin1']['b'].reshape(1, 1024))
    ws = [(p['W'], p['b'].reshape(1, -1)) for p in params['out']]
    return _head(g, ws)


# speedup vs baseline: 21.4572x; 1.0707x over previous
import functools

import jax
import jax.numpy as jnp
from jax import lax
from jax.experimental import pallas as pl
from jax.experimental.pallas import tpu as pltpu
from jax.experimental.pallas import tpu_sc as plsc

_INTERPRET = False

K = 30
KP = 32
B, N = 16, 2048
TN = 256
GS = 16
G = N // GS
ROWS = B * N
NW = 32
RPW = ROWS // NW
CH = 64
ECH = 1024
VP = 33


def _fold_bn(W, b, gamma, beta):
    s = gamma / jnp.sqrt(1.0 + 1e-5)
    return W * s[None, :], b * s + beta


def _scores_kernel(x_ref, xt_ref, wu_ref, bu_ref, wv_ref, e_ref,
                   s_ref, u_ref, v_ref, m_ref):
    xall = x_ref[0]
    xt = xt_ref[0]
    g = lax.dot_general(xt, xall, (((1,), (1,)), ((), ())),
                        preferred_element_type=jnp.float32)
    x2 = jnp.sum(xall * xall, axis=1)
    s = 2.0 * g - x2[None, :]
    s_ref[0] = s
    t = s
    for k in (1, 2, 4, 8):
        pad = jnp.full((TN, k), -jnp.inf, jnp.float32)
        t = jnp.maximum(t, jnp.concatenate([t[:, k:], pad], axis=1))
    m_ref[0] = jnp.dot(t, e_ref[...], preferred_element_type=jnp.float32)
    u_ref[0] = jnp.dot(xt, wu_ref[...], preferred_element_type=jnp.float32) + bu_ref[...]
    v_ref[0] = jnp.dot(xt, wv_ref[...], preferred_element_type=jnp.float32)


def _scores(x, wu, bu, wv):
    d = x.shape[-1]
    grid = (B, N // TN)
    ext = jnp.zeros((N, G), jnp.float32).at[
        16 * jnp.arange(G), jnp.arange(G)].set(1.0)
    return pl.pallas_call(
        _scores_kernel,
        grid=grid,
        in_specs=[
            pl.BlockSpec((1, N, d), lambda b, t: (b, 0, 0)),
            pl.BlockSpec((1, TN, d), lambda b, t: (b, t, 0)),
            pl.BlockSpec((d, 32), lambda b, t: (0, 0)),
            pl.BlockSpec((1, 32), lambda b, t: (0, 0)),
            pl.BlockSpec((d, 32), lambda b, t: (0, 0)),
            pl.BlockSpec((N, G), lambda b, t: (0, 0)),
        ],
        out_specs=[
            pl.BlockSpec((1, TN, N), lambda b, t: (b, t, 0)),
            pl.BlockSpec((1, TN, 32), lambda b, t: (b, t, 0)),
            pl.BlockSpec((1, TN, 32), lambda b, t: (b, t, 0)),
            pl.BlockSpec((1, TN, G), lambda b, t: (b, t, 0)),
        ],
        out_shape=[
            jax.ShapeDtypeStruct((B, N, N), jnp.float32),
            jax.ShapeDtypeStruct((B, N, 32), jnp.float32),
            jax.ShapeDtypeStruct((B, N, 32), jnp.float32),
            jax.ShapeDtypeStruct((B, N, G), jnp.float32),
        ],
        interpret=_INTERPRET,
    )(x, x, wu, bu, wv, ext)


def _sort16(k, v):
    return plsc.sort_key_val(k, v, descending=True)


def _merge16kv(ak, av, bk, bv):
    brk, brv = lax.rev(bk, (0,)), lax.rev(bv, (0,))
    m = ak >= brk
    hk = jnp.where(m, ak, brk)
    hv = jnp.where(m, av, brv)
    lk = jnp.where(m, brk, ak)
    lv = jnp.where(m, brv, av)
    hk, hv = _sort16(hk, hv)
    lk, lv = _sort16(lk, lv)
    return hk, hv, lk, lv


def _merge32kv(a, b):
    a0k, a0v, a1k, a1v = a
    b0k, b0v, b1k, b1v = b
    r1k, r1v = lax.rev(b1k, (0,)), lax.rev(b1v, (0,))
    r0k, r0v = lax.rev(b0k, (0,)), lax.rev(b0v, (0,))
    m0 = a0k >= r1k
    l0k = jnp.where(m0, a0k, r1k)
    l0v = jnp.where(m0, a0v, r1v)
    m1 = a1k >= r0k
    l1k = jnp.where(m1, a1k, r0k)
    l1v = jnp.where(m1, a1v, r0v)
    m2 = l0k >= l1k
    hk = jnp.where(m2, l0k, l1k)
    hv = jnp.where(m2, l0v, l1v)
    lk = jnp.where(m2, l1k, l0k)
    lv = jnp.where(m2, l1v, l0v)
    hk, hv = _sort16(hk, hv)
    lk, lv = _sort16(lk, lv)
    return hk, hv, lk, lv


def _top32_net(pairs):
    units = [_merge16kv(*pairs[2 * j], *pairs[2 * j + 1])
             for j in range(len(pairs) // 2)]
    while len(units) > 1:
        units = [_merge32kv(units[2 * j], units[2 * j + 1])
                 for j in range(len(units) // 2)]
    return units[0]


SCH = 16
NCH = RPW // SCH


def _topk_sc(s2, m1):
    mesh = plsc.VectorSubcoreMesh(core_axis_name="c", subcore_axis_name="s")

    @functools.partial(
        pl.kernel,
        mesh=mesh,
        out_type=jax.ShapeDtypeStruct((ROWS * KP,), jnp.int32),
        compiler_params=pltpu.CompilerParams(needs_layout_passes=False),
        interpret=_INTERPRET,
        scratch_types=[
            pltpu.VMEM((2, SCH, N), jnp.float32),
            pltpu.VMEM((2, SCH, G), jnp.float32),
            pltpu.VMEM((SCH * KP,), jnp.int32),
            pltpu.VMEM((4, KP), jnp.int32),
            pltpu.SemaphoreType.DMA,
            pltpu.SemaphoreType.DMA,
            pltpu.SemaphoreType.DMA,
            pltpu.SemaphoreType.DMA,
        ],
    )
    def topk_kernel(s_hbm, m_hbm, o_hbm, s_buf, m_buf, out_buf, idxv,
                    ss0, ss1, ms0, ms1):
        wid = lax.axis_index("s") * 2 + lax.axis_index("c")
        wbase = wid * RPW
        iot = lax.broadcasted_iota(jnp.int32, (GS,), 0)
        ssem = (ss0, ss1)
        msem = (ms0, ms1)

        def start(c, par):
            rb = pl.multiple_of(wbase + c * SCH, SCH)
            pltpu.make_async_copy(
                s_hbm.at[pl.ds(rb, SCH)], s_buf.at[par], ssem[par]).start()
            pltpu.make_async_copy(
                m_hbm.at[pl.ds(rb, SCH)], m_buf.at[par], msem[par]).start()

        def wait(par):
            pltpu.make_async_copy(
                s_hbm.at[pl.ds(0, SCH)], s_buf.at[par], ssem[par]).wait()
            pltpu.make_async_copy(
                m_hbm.at[pl.ds(0, SCH)], m_buf.at[par], msem[par]).wait()

        def compute(c, par):
            rbase = wbase + c * SCH

            def row_body(r, q, carry_r):
                rabs = rbase + r
                rsplat = jnp.full((GS,), r, jnp.int32)
                leaves = []
                for j in range(G // GS):
                    kj = plsc.load_gather(
                        m_buf.at[par], [rsplat, iot + (GS * j)])
                    leaves.append(_sort16(kj, iot + (GS * j)))
                _, ghv, _, glv = _top32_net(leaves)
                idxq = idxv.at[q]
                idxq[pl.ds(0, GS)] = ghv
                idxq[pl.ds(GS, GS)] = glv
                el = []
                for j in range(KP):
                    g = plsc.load_gather(idxq, [jnp.full((GS,), j, jnp.int32)])
                    col = g * GS + iot
                    kj = plsc.load_gather(s_buf.at[par], [rsplat, col])
                    el.append(_sort16(kj, col))
                _, hv, _, lv = _top32_net(el)
                oo = r * KP
                out_buf[pl.ds(oo, GS)] = hv
                pad = jnp.full((GS,), lax.rem(rabs, N), jnp.int32)
                lv = jnp.where(iot >= GS - 2, pad, lv)
                out_buf[pl.ds(oo + GS, GS)] = lv
                return carry_r

            def row_quad(i, carry_r):
                for q in range(4):
                    row_body(4 * i + q, q, carry_r)
                return carry_r

            lax.fori_loop(0, SCH // 4, row_quad, 0)
            pltpu.sync_copy(
                out_buf,
                o_hbm.at[pl.ds(pl.multiple_of(rbase * KP, SCH * KP), SCH * KP)])

        start(0, 0)

        def pair_body(t, carry):
            for par in range(2):
                c = 2 * t + par
                wait(par)

                @pl.when(c + 1 < NCH)
                def _():
                    start(c + 1, 1 - par)

                compute(c, par)
            return carry

        lax.fori_loop(0, NCH // 2, pair_body, 0)

    return topk_kernel(s2, m1)


def _gather_sc(v2, idx):
    mesh = plsc.VectorSubcoreMesh(core_axis_name="c", subcore_axis_name="s")
    epw = ROWS * KP // NW

    @functools.partial(
        pl.kernel,
        mesh=mesh,
        out_type=jax.ShapeDtypeStruct((ROWS * KP // 4, 128), jnp.float32),
        compiler_params=pltpu.CompilerParams(needs_layout_passes=False),
        interpret=_INTERPRET,
        scratch_types=[
            pltpu.VMEM((N * VP,), jnp.float32),
            pltpu.VMEM((ECH,), jnp.int32),
            pltpu.VMEM((ECH // 4, 128), jnp.float32),
        ],
    )
    def gather_kernel(v_hbm, i_hbm, o_hbm, vtab, ibuf, obuf):
        wid = lax.axis_index("s") * 2 + lax.axis_index("c")
        ebase = wid * epw
        b = wid // 2
        pltpu.sync_copy(
            v_hbm.at[pl.ds(pl.multiple_of(b * N * VP, N * VP), N * VP)], vtab)
        iot = lax.broadcasted_iota(jnp.int32, (GS,), 0)
        ec0 = (iot % 4) * 32
        iot4 = iot // 4

        def body(t, carry):
            off = ebase + t * ECH
            pltpu.sync_copy(
                i_hbm.at[pl.ds(pl.multiple_of(off, ECH), ECH)], ibuf)

            def edges16(e0, carry2):
                for ee in range(GS):
                    nid = plsc.load_gather(
                        ibuf, [jnp.full((GS,), e0 + ee, jnp.int32)]) * VP
                    r = jnp.full((GS,), e0 // 4 + ee // 4, jnp.int32)
                    lo = plsc.load_gather(vtab, [nid + iot])
                    hi = plsc.load_gather(vtab, [nid + iot + 16])
                    c0 = (ee % 4) * 32
                    plsc.store_scatter(obuf, [r, iot + c0], lo)
                    plsc.store_scatter(obuf, [r, iot + (c0 + 16)], hi)
                return carry2

            lax.fori_loop(0, ECH // GS, lambda i, c: edges16(i * GS, c), 0)
            pltpu.sync_copy(
                obuf, o_hbm.at[pl.ds(pl.multiple_of(off // 4, 256), ECH // 4)])
            return carry

        lax.fori_loop(0, epw // ECH, body, 0)

    return gather_kernel(v2, idx)


RP = TN * KP // 4


def _edge_kernel(u_ref, ve_ref, w2_ref, b2_ref, w3_ref, b3_ref, o_ref):
    u = u_ref[0]
    ve = ve_ref[0]
    u4 = jnp.tile(u, (1, 4))
    ub = jnp.broadcast_to(u4[:, None, :], (TN, KP // 4, 128)).reshape(RP, 128)
    h1 = jnp.maximum(ve + ub, 0.0)
    h2 = jnp.dot(h1, w2_ref[...], preferred_element_type=jnp.float32) + b2_ref[...]
    h2 = jnp.maximum(h2, 0.0)
    msg = jnp.dot(h2, w3_ref[...], preferred_element_type=jnp.float32) + b3_ref[...]
    t = jnp.max(msg.reshape(TN, KP // 4, 128), axis=1)
    o_ref[0] = jnp.maximum(
        jnp.maximum(t[:, 0:32], t[:, 32:64]),
        jnp.maximum(t[:, 64:96], t[:, 96:128]))


def _edge_mlp_max(u, ve, w2, b2, w3, b3):
    w2d = jnp.kron(jnp.eye(4, dtype=jnp.float32), w2)
    w3d = jnp.kron(jnp.eye(4, dtype=jnp.float32), w3)
    b2d = jnp.tile(b2, (1, 4))
    b3d = jnp.tile(b3, (1, 4))
    grid = (B, N // TN)
    return pl.pallas_call(
        _edge_kernel,
        grid=grid,
        in_specs=[
            pl.BlockSpec((1, TN, 32), lambda b, t: (b, t, 0)),
            pl.BlockSpec((1, RP, 128), lambda b, t: (b, t, 0)),
            pl.BlockSpec((128, 128), lambda b, t: (0, 0)),
            pl.BlockSpec((1, 128), lambda b, t: (0, 0)),
            pl.BlockSpec((128, 128), lambda b, t: (0, 0)),
            pl.BlockSpec((1, 128), lambda b, t: (0, 0)),
        ],
        out_specs=pl.BlockSpec((1, TN, 32), lambda b, t: (b, t, 0)),
        out_shape=jax.ShapeDtypeStruct((B, N, 32), jnp.float32),
        interpret=_INTERPRET,
    )(u, ve, w2d, b2d, w3d, b3d)


def _pool_kernel(x1_ref, x2_ref, x3_ref, w_ref, b_ref, o_ref):
    t = pl.program_id(1)
    h = jnp.concatenate([x1_ref[0], x2_ref[0], x3_ref[0]], axis=1)
    h = jnp.dot(h, w_ref[...], preferred_element_type=jnp.float32) + b_ref[...]
    m = jnp.max(h, axis=0, keepdims=True)[None]

    @pl.when(t == 0)
    def _():
        o_ref[...] = m

    @pl.when(t != 0)
    def _():
        o_ref[...] = jnp.maximum(o_ref[...], m)


def _pool(x1, x2, x3, w, b):
    grid = (B, N // TN)
    return pl.pallas_call(
        _pool_kernel,
        grid=grid,
        in_specs=[
            pl.BlockSpec((1, TN, 32), lambda b, t: (b, t, 0)),
            pl.BlockSpec((1, TN, 32), lambda b, t: (b, t, 0)),
            pl.BlockSpec((1, TN, 32), lambda b, t: (b, t, 0)),
            pl.BlockSpec((96, 1024), lambda b, t: (0, 0)),
            pl.BlockSpec((1, 1024), lambda b, t: (0, 0)),
        ],
        out_specs=pl.BlockSpec((1, 1, 1024), lambda b, t: (b, 0, 0)),
        out_shape=jax.ShapeDtypeStruct((B, 1, 1024), jnp.float32),
        interpret=_INTERPRET,
    )(x1, x2, x3, w, b).reshape(B, 1024)


def _head_kernel(g_ref, w0, b0, w1, b1, w2, b2, w3, b3, o_ref):
    g = g_ref[...]
    g = jnp.maximum(jnp.dot(g, w0[...], preferred_element_type=jnp.float32) + b0[...], 0.0)
    g = jnp.maximum(jnp.dot(g, w1[...], preferred_element_type=jnp.float32) + b1[...], 0.0)
    g = jnp.maximum(jnp.dot(g, w2[...], preferred_element_type=jnp.float32) + b2[...], 0.0)
    o_ref[...] = jnp.dot(g, w3[...], preferred_element_type=jnp.float32) + b3[...]


def _head(g, ws):
    ins = []
    specs = [pl.BlockSpec(g.shape, lambda: (0, 0))]
    for w, b in ws:
        ins += [w, b]
        specs += [pl.BlockSpec(w.shape, lambda: (0, 0)),
                  pl.BlockSpec(b.shape, lambda: (0, 0))]
    ncls = ws[-1][0].shape[1]
    return pl.pallas_call(
        _head_kernel,
        in_specs=specs,
        out_specs=pl.BlockSpec((B, ncls), lambda: (0, 0)),
        out_shape=jax.ShapeDtypeStruct((B, ncls), jnp.float32),
        interpret=_INTERPRET,
    )(g, *ins)


def _layer(x, layers):
    d = x.shape[-1]
    p1, p2, p3 = layers
    w1t, b1t = _fold_bn(p1['W'], p1['b'], p1['gamma'], p1['beta'])
    a, c = w1t[:d], w1t[d:]
    wu = a - c
    bu = b1t.reshape(1, 32)
    w2t, b2t = _fold_bn(p2['W'], p2['b'], p2['gamma'], p2['beta'])
    s, u, v, m = _scores(x, wu, bu, c)
    idxg = _topk_sc(s.reshape(ROWS, N), m.reshape(ROWS, G))
    vp = jnp.pad(v, ((0, 0), (0, 0), (0, VP - 32)))
    ve = _gather_sc(vp.reshape(B * N * VP), idxg)
    ve = ve.reshape(B, N * KP // 4, 128)
    return _edge_mlp_max(u, ve, w2t, b2t.reshape(1, 32),
                         p3['W'], p3['b'].reshape(1, 32))


def kernel(data, params):
    x = data
    xs = []
    for li in range(3):
        x = _layer(x, params['conv%d' % li])
        xs.append(x)
    g = _pool(xs[0], xs[1], xs[2], params['lin1']['W'],
              params['lin1']['b'].reshape(1, 1024))
    ws = [(p['W'], p['b'].reshape(1, -1)) for p in params['out']]
    return _head(g, ws)


# speedup vs baseline: 22.4853x; 1.0479x over previous
import functools

import jax
import jax.numpy as jnp
from jax import lax
from jax.experimental import pallas as pl
from jax.experimental.pallas import tpu as pltpu
from jax.experimental.pallas import tpu_sc as plsc

_INTERPRET = False

K = 30
KP = 32
B, N = 16, 2048
TN = 256
GS = 16
G = N // GS
ROWS = B * N
NW = 32
RPW = ROWS // NW
CH = 64
ECH = 1024
VP = 33


def _fold_bn(W, b, gamma, beta):
    s = gamma / jnp.sqrt(1.0 + 1e-5)
    return W * s[None, :], b * s + beta


def _scores_kernel(x_ref, xt_ref, wu_ref, bu_ref, wv_ref, e_ref,
                   s_ref, u_ref, v_ref, m_ref):
    xall = x_ref[0]
    xt = xt_ref[0]
    g = lax.dot_general(xt, xall, (((1,), (1,)), ((), ())),
                        preferred_element_type=jnp.float32)
    x2 = jnp.sum(xall * xall, axis=1)
    s = 2.0 * g - x2[None, :]
    s_ref[0] = s
    t = s
    for k in (1, 2, 4, 8):
        pad = jnp.full((TN, k), -jnp.inf, jnp.float32)
        t = jnp.maximum(t, jnp.concatenate([t[:, k:], pad], axis=1))
    m_ref[0] = jnp.dot(t, e_ref[...], preferred_element_type=jnp.float32)
    u_ref[0] = jnp.dot(xt, wu_ref[...], preferred_element_type=jnp.float32) + bu_ref[...]
    v_ref[0] = jnp.dot(xt, wv_ref[...], preferred_element_type=jnp.float32)


def _scores(x, wu, bu, wv):
    d = x.shape[-1]
    grid = (B, N // TN)
    ext = jnp.zeros((N, G), jnp.float32).at[
        16 * jnp.arange(G), jnp.arange(G)].set(1.0)
    return pl.pallas_call(
        _scores_kernel,
        grid=grid,
        in_specs=[
            pl.BlockSpec((1, N, d), lambda b, t: (b, 0, 0)),
            pl.BlockSpec((1, TN, d), lambda b, t: (b, t, 0)),
            pl.BlockSpec((d, 32), lambda b, t: (0, 0)),
            pl.BlockSpec((1, 32), lambda b, t: (0, 0)),
            pl.BlockSpec((d, 32), lambda b, t: (0, 0)),
            pl.BlockSpec((N, G), lambda b, t: (0, 0)),
        ],
        out_specs=[
            pl.BlockSpec((1, TN, N), lambda b, t: (b, t, 0)),
            pl.BlockSpec((1, TN, 32), lambda b, t: (b, t, 0)),
            pl.BlockSpec((1, TN, 32), lambda b, t: (b, t, 0)),
            pl.BlockSpec((1, TN, G), lambda b, t: (b, t, 0)),
        ],
        out_shape=[
            jax.ShapeDtypeStruct((B, N, N), jnp.float32),
            jax.ShapeDtypeStruct((B, N, 32), jnp.float32),
            jax.ShapeDtypeStruct((B, N, 32), jnp.float32),
            jax.ShapeDtypeStruct((B, N, G), jnp.float32),
        ],
        interpret=_INTERPRET,
    )(x, x, wu, bu, wv, ext)


def _sort16(k, v):
    return plsc.sort_key_val(k, v, descending=True)


def _merge16kv(ak, av, bk, bv):
    brk, brv = lax.rev(bk, (0,)), lax.rev(bv, (0,))
    m = ak >= brk
    hk = jnp.where(m, ak, brk)
    hv = jnp.where(m, av, brv)
    lk = jnp.where(m, brk, ak)
    lv = jnp.where(m, brv, av)
    hk, hv = _sort16(hk, hv)
    lk, lv = _sort16(lk, lv)
    return hk, hv, lk, lv


def _merge32kv(a, b):
    a0k, a0v, a1k, a1v = a
    b0k, b0v, b1k, b1v = b
    r1k, r1v = lax.rev(b1k, (0,)), lax.rev(b1v, (0,))
    r0k, r0v = lax.rev(b0k, (0,)), lax.rev(b0v, (0,))
    m0 = a0k >= r1k
    l0k = jnp.where(m0, a0k, r1k)
    l0v = jnp.where(m0, a0v, r1v)
    m1 = a1k >= r0k
    l1k = jnp.where(m1, a1k, r0k)
    l1v = jnp.where(m1, a1v, r0v)
    m2 = l0k >= l1k
    hk = jnp.where(m2, l0k, l1k)
    hv = jnp.where(m2, l0v, l1v)
    lk = jnp.where(m2, l1k, l0k)
    lv = jnp.where(m2, l1v, l0v)
    hk, hv = _sort16(hk, hv)
    lk, lv = _sort16(lk, lv)
    return hk, hv, lk, lv


def _top32_net(pairs):
    units = [_merge16kv(*pairs[2 * j], *pairs[2 * j + 1])
             for j in range(len(pairs) // 2)]
    while len(units) > 1:
        units = [_merge32kv(units[2 * j], units[2 * j + 1])
                 for j in range(len(units) // 2)]
    return units[0]


SCH = 16
NCH = RPW // SCH


def _topk_sc(s2, m1):
    mesh = plsc.VectorSubcoreMesh(core_axis_name="c", subcore_axis_name="s")

    @functools.partial(
        pl.kernel,
        mesh=mesh,
        out_type=jax.ShapeDtypeStruct((ROWS * KP,), jnp.int32),
        compiler_params=pltpu.CompilerParams(needs_layout_passes=False),
        interpret=_INTERPRET,
        scratch_types=[
            pltpu.VMEM((2, SCH, N), jnp.float32),
            pltpu.VMEM((2, SCH, G), jnp.float32),
            pltpu.VMEM((SCH * KP,), jnp.int32),
            pltpu.VMEM((4, KP), jnp.int32),
            pltpu.SemaphoreType.DMA,
            pltpu.SemaphoreType.DMA,
            pltpu.SemaphoreType.DMA,
            pltpu.SemaphoreType.DMA,
        ],
    )
    def topk_kernel(s_hbm, m_hbm, o_hbm, s_buf, m_buf, out_buf, idxv,
                    ss0, ss1, ms0, ms1):
        wid = lax.axis_index("s") * 2 + lax.axis_index("c")
        wbase = wid * RPW
        iot = lax.broadcasted_iota(jnp.int32, (GS,), 0)
        ssem = (ss0, ss1)
        msem = (ms0, ms1)

        def start(c, par):
            rb = pl.multiple_of(wbase + c * SCH, SCH)
            pltpu.make_async_copy(
                s_hbm.at[pl.ds(rb, SCH)], s_buf.at[par], ssem[par]).start()
            pltpu.make_async_copy(
                m_hbm.at[pl.ds(rb, SCH)], m_buf.at[par], msem[par]).start()

        def wait(par):
            pltpu.make_async_copy(
                s_hbm.at[pl.ds(0, SCH)], s_buf.at[par], ssem[par]).wait()
            pltpu.make_async_copy(
                m_hbm.at[pl.ds(0, SCH)], m_buf.at[par], msem[par]).wait()

        def compute(c, par):
            rbase = wbase + c * SCH

            def row_body(r, q, carry_r):
                rabs = rbase + r
                rsplat = jnp.full((GS,), r, jnp.int32)
                leaves = []
                for j in range(G // GS):
                    kj = plsc.load_gather(
                        m_buf.at[par], [rsplat, iot + (GS * j)])
                    leaves.append(_sort16(kj, iot + (GS * j)))
                _, ghv, _, glv = _top32_net(leaves)
                idxq = idxv.at[q]
                idxq[pl.ds(0, GS)] = ghv
                idxq[pl.ds(GS, GS)] = glv
                el = []
                for j in range(KP):
                    g = plsc.load_gather(idxq, [jnp.full((GS,), j, jnp.int32)])
                    col = g * GS + iot
                    kj = plsc.load_gather(s_buf.at[par], [rsplat, col])
                    el.append(_sort16(kj, col))
                _, hv, _, lv = _top32_net(el)
                oo = r * KP
                out_buf[pl.ds(oo, GS)] = hv
                pad = jnp.full((GS,), lax.rem(rabs, N), jnp.int32)
                lv = jnp.where(iot >= GS - 2, pad, lv)
                out_buf[pl.ds(oo + GS, GS)] = lv
                return carry_r

            def row_quad(i, carry_r):
                for q in range(4):
                    row_body(4 * i + q, q, carry_r)
                return carry_r

            lax.fori_loop(0, SCH // 4, row_quad, 0)
            pltpu.sync_copy(
                out_buf,
                o_hbm.at[pl.ds(pl.multiple_of(rbase * KP, SCH * KP), SCH * KP)])

        start(0, 0)

        def pair_body(t, carry):
            for par in range(2):
                c = 2 * t + par
                wait(par)

                @pl.when(c + 1 < NCH)
                def _():
                    start(c + 1, 1 - par)

                compute(c, par)
            return carry

        lax.fori_loop(0, NCH // 2, pair_body, 0)

    return topk_kernel(s2, m1)


def _gather_sc(v2, idx):
    mesh = plsc.VectorSubcoreMesh(core_axis_name="c", subcore_axis_name="s")
    epw = ROWS * KP // NW

    @functools.partial(
        pl.kernel,
        mesh=mesh,
        out_type=jax.ShapeDtypeStruct((ROWS * KP // 4, 128), jnp.float32),
        compiler_params=pltpu.CompilerParams(needs_layout_passes=False),
        interpret=_INTERPRET,
        scratch_types=[
            pltpu.VMEM((N * VP,), jnp.float32),
            pltpu.VMEM((ECH,), jnp.int32),
            pltpu.VMEM((ECH // 4, 128), jnp.float32),
        ],
    )
    def gather_kernel(v_hbm, i_hbm, o_hbm, vtab, ibuf, obuf):
        wid = lax.axis_index("s") * 2 + lax.axis_index("c")
        ebase = wid * epw
        b = wid // 2
        pltpu.sync_copy(
            v_hbm.at[pl.ds(pl.multiple_of(b * N * VP, N * VP), N * VP)], vtab)
        iot = lax.broadcasted_iota(jnp.int32, (GS,), 0)
        ec0 = (iot % 4) * 32
        iot4 = iot // 4

        def body(t, carry):
            off = ebase + t * ECH
            pltpu.sync_copy(
                i_hbm.at[pl.ds(pl.multiple_of(off, ECH), ECH)], ibuf)

            def edges16(e0, carry2):
                nids = ibuf[pl.ds(e0, GS)] * VP
                erow = (e0 // 4) + iot4
                for c in range(32):
                    fidx = (iot + c) & 31
                    vals = plsc.load_gather(vtab, [nids + fidx])
                    plsc.store_scatter(obuf, [erow, ec0 + fidx], vals)
                return carry2

            lax.fori_loop(0, ECH // GS, lambda i, c: edges16(i * GS, c), 0)
            pltpu.sync_copy(
                obuf, o_hbm.at[pl.ds(pl.multiple_of(off // 4, 256), ECH // 4)])
            return carry

        lax.fori_loop(0, epw // ECH, body, 0)

    return gather_kernel(v2, idx)


RP = TN * KP // 4


def _edge_kernel(u_ref, ve_ref, w2_ref, b2_ref, w3_ref, b3_ref, o_ref):
    u = u_ref[0]
    ve = ve_ref[0]
    u4 = jnp.tile(u, (1, 4))
    ub = jnp.broadcast_to(u4[:, None, :], (TN, KP // 4, 128)).reshape(RP, 128)
    h1 = jnp.maximum(ve + ub, 0.0)
    h2 = jnp.dot(h1, w2_ref[...], preferred_element_type=jnp.float32) + b2_ref[...]
    h2 = jnp.maximum(h2, 0.0)
    msg = jnp.dot(h2, w3_ref[...], preferred_element_type=jnp.float32) + b3_ref[...]
    t = jnp.max(msg.reshape(TN, KP // 4, 128), axis=1)
    o_ref[0] = jnp.maximum(
        jnp.maximum(t[:, 0:32], t[:, 32:64]),
        jnp.maximum(t[:, 64:96], t[:, 96:128]))


def _edge_mlp_max(u, ve, w2, b2, w3, b3):
    w2d = jnp.kron(jnp.eye(4, dtype=jnp.float32), w2)
    w3d = jnp.kron(jnp.eye(4, dtype=jnp.float32), w3)
    b2d = jnp.tile(b2, (1, 4))
    b3d = jnp.tile(b3, (1, 4))
    grid = (B, N // TN)
    return pl.pallas_call(
        _edge_kernel,
        grid=grid,
        in_specs=[
            pl.BlockSpec((1, TN, 32), lambda b, t: (b, t, 0)),
            pl.BlockSpec((1, RP, 128), lambda b, t: (b, t, 0)),
            pl.BlockSpec((128, 128), lambda b, t: (0, 0)),
            pl.BlockSpec((1, 128), lambda b, t: (0, 0)),
            pl.BlockSpec((128, 128), lambda b, t: (0, 0)),
            pl.BlockSpec((1, 128), lambda b, t: (0, 0)),
        ],
        out_specs=pl.BlockSpec((1, TN, 32), lambda b, t: (b, t, 0)),
        out_shape=jax.ShapeDtypeStruct((B, N, 32), jnp.float32),
        interpret=_INTERPRET,
    )(u, ve, w2d, b2d, w3d, b3d)


def _pool_kernel(x1_ref, x2_ref, x3_ref, w_ref, b_ref, o_ref):
    t = pl.program_id(1)
    h = jnp.concatenate([x1_ref[0], x2_ref[0], x3_ref[0]], axis=1)
    h = jnp.dot(h, w_ref[...], preferred_element_type=jnp.float32) + b_ref[...]
    m = jnp.max(h, axis=0, keepdims=True)[None]

    @pl.when(t == 0)
    def _():
        o_ref[...] = m

    @pl.when(t != 0)
    def _():
        o_ref[...] = jnp.maximum(o_ref[...], m)


def _pool(x1, x2, x3, w, b):
    grid = (B, N // TN)
    return pl.pallas_call(
        _pool_kernel,
        grid=grid,
        in_specs=[
            pl.BlockSpec((1, TN, 32), lambda b, t: (b, t, 0)),
            pl.BlockSpec((1, TN, 32), lambda b, t: (b, t, 0)),
            pl.BlockSpec((1, TN, 32), lambda b, t: (b, t, 0)),
            pl.BlockSpec((96, 1024), lambda b, t: (0, 0)),
            pl.BlockSpec((1, 1024), lambda b, t: (0, 0)),
        ],
        out_specs=pl.BlockSpec((1, 1, 1024), lambda b, t: (b, 0, 0)),
        out_shape=jax.ShapeDtypeStruct((B, 1, 1024), jnp.float32),
        interpret=_INTERPRET,
    )(x1, x2, x3, w, b).reshape(B, 1024)


def _head_kernel(g_ref, w0, b0, w1, b1, w2, b2, w3, b3, o_ref):
    g = g_ref[...]
    g = jnp.maximum(jnp.dot(g, w0[...], preferred_element_type=jnp.float32) + b0[...], 0.0)
    g = jnp.maximum(jnp.dot(g, w1[...], preferred_element_type=jnp.float32) + b1[...], 0.0)
    g = jnp.maximum(jnp.dot(g, w2[...], preferred_element_type=jnp.float32) + b2[...], 0.0)
    o_ref[...] = jnp.dot(g, w3[...], preferred_element_type=jnp.float32) + b3[...]


def _head(g, ws):
    ins = []
    specs = [pl.BlockSpec(g.shape, lambda: (0, 0))]
    for w, b in ws:
        ins += [w, b]
        specs += [pl.BlockSpec(w.shape, lambda: (0, 0)),
                  pl.BlockSpec(b.shape, lambda: (0, 0))]
    ncls = ws[-1][0].shape[1]
    return pl.pallas_call(
        _head_kernel,
        in_specs=specs,
        out_specs=pl.BlockSpec((B, ncls), lambda: (0, 0)),
        out_shape=jax.ShapeDtypeStruct((B, ncls), jnp.float32),
        interpret=_INTERPRET,
    )(g, *ins)


def _layer(x, layers):
    d = x.shape[-1]
    p1, p2, p3 = layers
    w1t, b1t = _fold_bn(p1['W'], p1['b'], p1['gamma'], p1['beta'])
    a, c = w1t[:d], w1t[d:]
    wu = a - c
    bu = b1t.reshape(1, 32)
    w2t, b2t = _fold_bn(p2['W'], p2['b'], p2['gamma'], p2['beta'])
    s, u, v, m = _scores(x, wu, bu, c)
    idxg = _topk_sc(s.reshape(ROWS, N), m.reshape(ROWS, G))
    vp = jnp.pad(v, ((0, 0), (0, 0), (0, VP - 32)))
    ve = _gather_sc(vp.reshape(B * N * VP), idxg)
    ve = ve.reshape(B, N * KP // 4, 128)
    return _edge_mlp_max(u, ve, w2t, b2t.reshape(1, 32),
                         p3['W'], p3['b'].reshape(1, 32))


def kernel(data, params):
    x = data
    xs = []
    for li in range(3):
        x = _layer(x, params['conv%d' % li])
        xs.append(x)
    g = _pool(xs[0], xs[1], xs[2], params['lin1']['W'],
              params['lin1']['b'].reshape(1, 1024))
    ws = [(p['W'], p['b'].reshape(1, -1)) for p in params['out']]
    return _head(g, ws)


# speedup vs baseline: 24.9303x; 1.1087x over previous
import functools

import jax
import jax.numpy as jnp
from jax import lax
from jax.experimental import pallas as pl
from jax.experimental.pallas import tpu as pltpu
from jax.experimental.pallas import tpu_sc as plsc

_INTERPRET = False

K = 30
KP = 32
B, N = 16, 2048
TN = 256
GS = 16
G = N // GS
ROWS = B * N
NW = 32
RPW = ROWS // NW
CH = 64
ECH = 1024
VP = 33


def _fold_bn(W, b, gamma, beta):
    s = gamma / jnp.sqrt(1.0 + 1e-5)
    return W * s[None, :], b * s + beta


def _scores_kernel(x_ref, xt_ref, wu_ref, bu_ref, wv_ref, e_ref,
                   s_ref, u_ref, v_ref, m_ref):
    xall = x_ref[0]
    xt = xt_ref[0]
    g = lax.dot_general(xt, xall, (((1,), (1,)), ((), ())),
                        preferred_element_type=jnp.float32)
    x2 = jnp.sum(xall * xall, axis=1)
    s = 2.0 * g - x2[None, :]
    s_ref[0] = s
    t = s
    for k in (1, 2, 4, 8):
        pad = jnp.full((TN, k), -jnp.inf, jnp.float32)
        t = jnp.maximum(t, jnp.concatenate([t[:, k:], pad], axis=1))
    m_ref[0] = jnp.dot(t, e_ref[...], preferred_element_type=jnp.float32)
    u_ref[0] = jnp.dot(xt, wu_ref[...], preferred_element_type=jnp.float32) + bu_ref[...]
    v_ref[0] = jnp.dot(xt, wv_ref[...], preferred_element_type=jnp.float32)


def _scores(x, wu, bu, wv):
    d = x.shape[-1]
    grid = (B, N // TN)
    ext = jnp.zeros((N, G), jnp.float32).at[
        16 * jnp.arange(G), jnp.arange(G)].set(1.0)
    return pl.pallas_call(
        _scores_kernel,
        grid=grid,
        in_specs=[
            pl.BlockSpec((1, N, d), lambda b, t: (b, 0, 0)),
            pl.BlockSpec((1, TN, d), lambda b, t: (b, t, 0)),
            pl.BlockSpec((d, 32), lambda b, t: (0, 0)),
            pl.BlockSpec((1, 32), lambda b, t: (0, 0)),
            pl.BlockSpec((d, 32), lambda b, t: (0, 0)),
            pl.BlockSpec((N, G), lambda b, t: (0, 0)),
        ],
        out_specs=[
            pl.BlockSpec((1, TN, N), lambda b, t: (b, t, 0)),
            pl.BlockSpec((1, TN, 32), lambda b, t: (b, t, 0)),
            pl.BlockSpec((1, TN, 32), lambda b, t: (b, t, 0)),
            pl.BlockSpec((1, TN, G), lambda b, t: (b, t, 0)),
        ],
        out_shape=[
            jax.ShapeDtypeStruct((B, N, N), jnp.float32),
            jax.ShapeDtypeStruct((B, N, 32), jnp.float32),
            jax.ShapeDtypeStruct((B, N, 32), jnp.float32),
            jax.ShapeDtypeStruct((B, N, G), jnp.float32),
        ],
        interpret=_INTERPRET,
    )(x, x, wu, bu, wv, ext)


def _sort16(k, v, desc=True):
    return plsc.sort_key_val(k, v, descending=desc)


def _out32(hk, hv, lk, lv, out_desc):
    if out_desc:
        hk, hv = _sort16(hk, hv, True)
        lk, lv = _sort16(lk, lv, True)
        return hk, hv, lk, lv
    hk, hv = _sort16(hk, hv, False)
    lk, lv = _sort16(lk, lv, False)
    return lk, lv, hk, hv


def _merge16kv(ak, av, bk, bv, out_desc=True):
    m = ak >= bk
    hk = jnp.where(m, ak, bk)
    hv = jnp.where(m, av, bv)
    lk = jnp.where(m, bk, ak)
    lv = jnp.where(m, bv, av)
    return _out32(hk, hv, lk, lv, out_desc)


def _merge32kv(a, b, out_desc=True):
    a0k, a0v, a1k, a1v = a
    b0k, b0v, b1k, b1v = b
    m0 = a0k >= b0k
    l0k = jnp.where(m0, a0k, b0k)
    l0v = jnp.where(m0, a0v, b0v)
    m1 = a1k >= b1k
    l1k = jnp.where(m1, a1k, b1k)
    l1v = jnp.where(m1, a1v, b1v)
    m2 = l0k >= l1k
    hk = jnp.where(m2, l0k, l1k)
    hv = jnp.where(m2, l0v, l1v)
    lk = jnp.where(m2, l1k, l0k)
    lv = jnp.where(m2, l1v, l0v)
    return _out32(hk, hv, lk, lv, out_desc)


def _top32_net(pairs):
    units = [_merge16kv(*pairs[2 * j], *pairs[2 * j + 1],
                        out_desc=(j % 2 == 0))
             for j in range(len(pairs) // 2)]
    while len(units) > 1:
        units = [_merge32kv(units[2 * j], units[2 * j + 1],
                            out_desc=(j % 2 == 0))
                 for j in range(len(units) // 2)]
    return units[0]


SCH = 16
NCH = RPW // SCH


def _topk_sc(s2, m1):
    mesh = plsc.VectorSubcoreMesh(core_axis_name="c", subcore_axis_name="s")

    @functools.partial(
        pl.kernel,
        mesh=mesh,
        out_type=jax.ShapeDtypeStruct((ROWS * KP,), jnp.int32),
        compiler_params=pltpu.CompilerParams(needs_layout_passes=False),
        interpret=_INTERPRET,
        scratch_types=[
            pltpu.VMEM((2, SCH, N), jnp.float32),
            pltpu.VMEM((2, SCH, G), jnp.float32),
            pltpu.VMEM((SCH * KP,), jnp.int32),
            pltpu.VMEM((4, KP), jnp.int32),
            pltpu.SemaphoreType.DMA,
            pltpu.SemaphoreType.DMA,
            pltpu.SemaphoreType.DMA,
            pltpu.SemaphoreType.DMA,
        ],
    )
    def topk_kernel(s_hbm, m_hbm, o_hbm, s_buf, m_buf, out_buf, idxv,
                    ss0, ss1, ms0, ms1):
        wid = lax.axis_index("s") * 2 + lax.axis_index("c")
        wbase = wid * RPW
        iot = lax.broadcasted_iota(jnp.int32, (GS,), 0)
        ssem = (ss0, ss1)
        msem = (ms0, ms1)

        def start(c, par):
            rb = pl.multiple_of(wbase + c * SCH, SCH)
            pltpu.make_async_copy(
                s_hbm.at[pl.ds(rb, SCH)], s_buf.at[par], ssem[par]).start()
            pltpu.make_async_copy(
                m_hbm.at[pl.ds(rb, SCH)], m_buf.at[par], msem[par]).start()

        def wait(par):
            pltpu.make_async_copy(
                s_hbm.at[pl.ds(0, SCH)], s_buf.at[par], ssem[par]).wait()
            pltpu.make_async_copy(
                m_hbm.at[pl.ds(0, SCH)], m_buf.at[par], msem[par]).wait()

        def compute(c, par):
            rbase = wbase + c * SCH

            def row_body(r, q, carry_r):
                rabs = rbase + r
                rsplat = jnp.full((GS,), r, jnp.int32)
                leaves = []
                for j in range(G // GS):
                    kj = plsc.load_gather(
                        m_buf.at[par], [rsplat, iot + (GS * j)])
                    leaves.append(_sort16(kj, iot + (GS * j), j % 2 == 0))
                _, ghv, _, glv = _top32_net(leaves)
                idxq = idxv.at[q]
                idxq[pl.ds(0, GS)] = ghv
                idxq[pl.ds(GS, GS)] = glv
                el = []
                for j in range(KP):
                    g = plsc.load_gather(idxq, [jnp.full((GS,), j, jnp.int32)])
                    col = g * GS + iot
                    kj = plsc.load_gather(s_buf.at[par], [rsplat, col])
                    el.append(_sort16(kj, col, j % 2 == 0))
                _, hv, _, lv = _top32_net(el)
                oo = r * KP
                out_buf[pl.ds(oo, GS)] = hv
                pad = jnp.full((GS,), lax.rem(rabs, N), jnp.int32)
                lv = jnp.where(iot >= GS - 2, pad, lv)
                out_buf[pl.ds(oo + GS, GS)] = lv
                return carry_r

            def row_quad(i, carry_r):
                for q in range(4):
                    row_body(4 * i + q, q, carry_r)
                return carry_r

            lax.fori_loop(0, SCH // 4, row_quad, 0)
            pltpu.sync_copy(
                out_buf,
                o_hbm.at[pl.ds(pl.multiple_of(rbase * KP, SCH * KP), SCH * KP)])

        start(0, 0)

        def pair_body(t, carry):
            for par in range(2):
                c = 2 * t + par
                wait(par)

                @pl.when(c + 1 < NCH)
                def _():
                    start(c + 1, 1 - par)

                compute(c, par)
            return carry

        lax.fori_loop(0, NCH // 2, pair_body, 0)

    return topk_kernel(s2, m1)


def _gather_sc(v2, idx):
    mesh = plsc.VectorSubcoreMesh(core_axis_name="c", subcore_axis_name="s")
    epw = ROWS * KP // NW

    @functools.partial(
        pl.kernel,
        mesh=mesh,
        out_type=jax.ShapeDtypeStruct((ROWS * KP // 4, 128), jnp.float32),
        compiler_params=pltpu.CompilerParams(needs_layout_passes=False),
        interpret=_INTERPRET,
        scratch_types=[
            pltpu.VMEM((N * VP,), jnp.float32),
            pltpu.VMEM((ECH,), jnp.int32),
            pltpu.VMEM((ECH // 4, 128), jnp.float32),
        ],
    )
    def gather_kernel(v_hbm, i_hbm, o_hbm, vtab, ibuf, obuf):
        wid = lax.axis_index("s") * 2 + lax.axis_index("c")
        ebase = wid * epw
        b = wid // 2
        pltpu.sync_copy(
            v_hbm.at[pl.ds(pl.multiple_of(b * N * VP, N * VP), N * VP)], vtab)
        iot = lax.broadcasted_iota(jnp.int32, (GS,), 0)
        ec0 = (iot % 4) * 32
        iot4 = iot // 4

        def body(t, carry):
            off = ebase + t * ECH
            pltpu.sync_copy(
                i_hbm.at[pl.ds(pl.multiple_of(off, ECH), ECH)], ibuf)

            def edges16(e0, carry2):
                nids = ibuf[pl.ds(e0, GS)] * VP
                erow = (e0 // 4) + iot4
                for c in range(32):
                    fidx = (iot + c) & 31
                    vals = plsc.load_gather(vtab, [nids + fidx])
                    plsc.store_scatter(obuf, [erow, ec0 + fidx], vals)
                return carry2

            def edges32(i, c):
                edges16(i * 2 * GS, c)
                edges16(i * 2 * GS + GS, c)
                return c

            lax.fori_loop(0, ECH // GS // 2, edges32, 0)
            pltpu.sync_copy(
                obuf, o_hbm.at[pl.ds(pl.multiple_of(off // 4, 256), ECH // 4)])
            return carry

        lax.fori_loop(0, epw // ECH, body, 0)

    return gather_kernel(v2, idx)


RP = TN * KP // 4


def _edge_kernel(u_ref, ve_ref, w2_ref, b2_ref, w3_ref, b3_ref, o_ref):
    u = u_ref[0]
    ve = ve_ref[0]
    u4 = jnp.tile(u, (1, 4))
    ub = jnp.broadcast_to(u4[:, None, :], (TN, KP // 4, 128)).reshape(RP, 128)
    h1 = jnp.maximum(ve + ub, 0.0)
    h2 = jnp.dot(h1, w2_ref[...], preferred_element_type=jnp.float32) + b2_ref[...]
    h2 = jnp.maximum(h2, 0.0)
    msg = jnp.dot(h2, w3_ref[...], preferred_element_type=jnp.float32) + b3_ref[...]
    t = jnp.max(msg.reshape(TN, KP // 4, 128), axis=1)
    o_ref[0] = jnp.maximum(
        jnp.maximum(t[:, 0:32], t[:, 32:64]),
        jnp.maximum(t[:, 64:96], t[:, 96:128]))


def _edge_mlp_max(u, ve, w2, b2, w3, b3):
    w2d = jnp.kron(jnp.eye(4, dtype=jnp.float32), w2)
    w3d = jnp.kron(jnp.eye(4, dtype=jnp.float32), w3)
    b2d = jnp.tile(b2, (1, 4))
    b3d = jnp.tile(b3, (1, 4))
    grid = (B, N // TN)
    return pl.pallas_call(
        _edge_kernel,
        grid=grid,
        in_specs=[
            pl.BlockSpec((1, TN, 32), lambda b, t: (b, t, 0)),
            pl.BlockSpec((1, RP, 128), lambda b, t: (b, t, 0)),
            pl.BlockSpec((128, 128), lambda b, t: (0, 0)),
            pl.BlockSpec((1, 128), lambda b, t: (0, 0)),
            pl.BlockSpec((128, 128), lambda b, t: (0, 0)),
            pl.BlockSpec((1, 128), lambda b, t: (0, 0)),
        ],
        out_specs=pl.BlockSpec((1, TN, 32), lambda b, t: (b, t, 0)),
        out_shape=jax.ShapeDtypeStruct((B, N, 32), jnp.float32),
        interpret=_INTERPRET,
    )(u, ve, w2d, b2d, w3d, b3d)


def _pool_kernel(x1_ref, x2_ref, x3_ref, w_ref, b_ref, o_ref):
    t = pl.program_id(1)
    h = jnp.concatenate([x1_ref[0], x2_ref[0], x3_ref[0]], axis=1)
    h = jnp.dot(h, w_ref[...], preferred_element_type=jnp.float32) + b_ref[...]
    m = jnp.max(h, axis=0, keepdims=True)[None]

    @pl.when(t == 0)
    def _():
        o_ref[...] = m

    @pl.when(t != 0)
    def _():
        o_ref[...] = jnp.maximum(o_ref[...], m)


def _pool(x1, x2, x3, w, b):
    grid = (B, N // TN)
    return pl.pallas_call(
        _pool_kernel,
        grid=grid,
        in_specs=[
            pl.BlockSpec((1, TN, 32), lambda b, t: (b, t, 0)),
            pl.BlockSpec((1, TN, 32), lambda b, t: (b, t, 0)),
            pl.BlockSpec((1, TN, 32), lambda b, t: (b, t, 0)),
            pl.BlockSpec((96, 1024), lambda b, t: (0, 0)),
            pl.BlockSpec((1, 1024), lambda b, t: (0, 0)),
        ],
        out_specs=pl.BlockSpec((1, 1, 1024), lambda b, t: (b, 0, 0)),
        out_shape=jax.ShapeDtypeStruct((B, 1, 1024), jnp.float32),
        interpret=_INTERPRET,
    )(x1, x2, x3, w, b).reshape(B, 1024)


def _head_kernel(g_ref, w0, b0, w1, b1, w2, b2, w3, b3, o_ref):
    g = g_ref[...]
    g = jnp.maximum(jnp.dot(g, w0[...], preferred_element_type=jnp.float32) + b0[...], 0.0)
    g = jnp.maximum(jnp.dot(g, w1[...], preferred_element_type=jnp.float32) + b1[...], 0.0)
    g = jnp.maximum(jnp.dot(g, w2[...], preferred_element_type=jnp.float32) + b2[...], 0.0)
    o_ref[...] = jnp.dot(g, w3[...], preferred_element_type=jnp.float32) + b3[...]


def _head(g, ws):
    ins = []
    specs = [pl.BlockSpec(g.shape, lambda: (0, 0))]
    for w, b in ws:
        ins += [w, b]
        specs += [pl.BlockSpec(w.shape, lambda: (0, 0)),
                  pl.BlockSpec(b.shape, lambda: (0, 0))]
    ncls = ws[-1][0].shape[1]
    return pl.pallas_call(
        _head_kernel,
        in_specs=specs,
        out_specs=pl.BlockSpec((B, ncls), lambda: (0, 0)),
        out_shape=jax.ShapeDtypeStruct((B, ncls), jnp.float32),
        interpret=_INTERPRET,
    )(g, *ins)


def _layer(x, layers):
    d = x.shape[-1]
    p1, p2, p3 = layers
    w1t, b1t = _fold_bn(p1['W'], p1['b'], p1['gamma'], p1['beta'])
    a, c = w1t[:d], w1t[d:]
    wu = a - c
    bu = b1t.reshape(1, 32)
    w2t, b2t = _fold_bn(p2['W'], p2['b'], p2['gamma'], p2['beta'])
    s, u, v, m = _scores(x, wu, bu, c)
    idxg = _topk_sc(s.reshape(ROWS, N), m.reshape(ROWS, G))
    vp = jnp.pad(v, ((0, 0), (0, 0), (0, VP - 32)))
    ve = _gather_sc(vp.reshape(B * N * VP), idxg)
    ve = ve.reshape(B, N * KP // 4, 128)
    return _edge_mlp_max(u, ve, w2t, b2t.reshape(1, 32),
                         p3['W'], p3['b'].reshape(1, 32))


def kernel(data, params):
    x = data
    xs = []
    for li in range(3):
        x = _layer(x, params['conv%d' % li])
        xs.append(x)
    g = _pool(xs[0], xs[1], xs[2], params['lin1']['W'],
              params['lin1']['b'].reshape(1, 1024))
    ws = [(p['W'], p['b'].reshape(1, -1)) for p in params['out']]
    return _head(g, ws)


# speedup vs baseline: 29.0608x; 1.1657x over previous
import functools

import jax
import jax.numpy as jnp
from jax import lax
from jax.experimental import pallas as pl
from jax.experimental.pallas import tpu as pltpu
from jax.experimental.pallas import tpu_sc as plsc

_INTERPRET = False

K = 30
KP = 32
B, N = 16, 2048
TN = 256
GS = 16
G = N // GS
ROWS = B * N
NW = 32
RPW = ROWS // NW
CH = 64
ECH = 1024
VP = 33


def _fold_bn(W, b, gamma, beta):
    s = gamma / jnp.sqrt(1.0 + 1e-5)
    return W * s[None, :], b * s + beta


def _scores_kernel(x_ref, xt_ref, wu_ref, bu_ref, wv_ref, e_ref,
                   s_ref, u_ref, v_ref, m_ref):
    xall = x_ref[0]
    xt = xt_ref[0]
    g = lax.dot_general(xt, xall, (((1,), (1,)), ((), ())),
                        preferred_element_type=jnp.float32)
    x2 = jnp.sum(xall * xall, axis=1)
    s = 2.0 * g - x2[None, :]
    s_ref[0] = s
    t = s
    for k in (1, 2, 4, 8):
        pad = jnp.full((TN, k), -jnp.inf, jnp.float32)
        t = jnp.maximum(t, jnp.concatenate([t[:, k:], pad], axis=1))
    m_ref[0] = jnp.dot(t, e_ref[...], preferred_element_type=jnp.float32)
    u_ref[0] = jnp.dot(xt, wu_ref[...], preferred_element_type=jnp.float32) + bu_ref[...]
    v_ref[0] = jnp.dot(xt, wv_ref[...], preferred_element_type=jnp.float32)


def _scores(x, wu, bu, wv):
    d = x.shape[-1]
    grid = (B, N // TN)
    ext = jnp.zeros((N, G), jnp.float32).at[
        16 * jnp.arange(G), jnp.arange(G)].set(1.0)
    return pl.pallas_call(
        _scores_kernel,
        grid=grid,
        in_specs=[
            pl.BlockSpec((1, N, d), lambda b, t: (b, 0, 0)),
            pl.BlockSpec((1, TN, d), lambda b, t: (b, t, 0)),
            pl.BlockSpec((d, 32), lambda b, t: (0, 0)),
            pl.BlockSpec((1, 32), lambda b, t: (0, 0)),
            pl.BlockSpec((d, 32), lambda b, t: (0, 0)),
            pl.BlockSpec((N, G), lambda b, t: (0, 0)),
        ],
        out_specs=[
            pl.BlockSpec((1, TN, N), lambda b, t: (b, t, 0)),
            pl.BlockSpec((1, TN, 32), lambda b, t: (b, t, 0)),
            pl.BlockSpec((1, TN, 32), lambda b, t: (b, t, 0)),
            pl.BlockSpec((1, TN, G), lambda b, t: (b, t, 0)),
        ],
        out_shape=[
            jax.ShapeDtypeStruct((B, N, N), jnp.float32),
            jax.ShapeDtypeStruct((B, N, 32), jnp.float32),
            jax.ShapeDtypeStruct((B, N, 32), jnp.float32),
            jax.ShapeDtypeStruct((B, N, G), jnp.float32),
        ],
        interpret=_INTERPRET,
    )(x, x, wu, bu, wv, ext)


def _sort16(k, v, desc=True):
    return plsc.sort_key_val(k, v, descending=desc)


def _out32(hk, hv, lk, lv, out_desc):
    if out_desc:
        hk, hv = _sort16(hk, hv, True)
        lk, lv = _sort16(lk, lv, True)
        return hk, hv, lk, lv
    hk, hv = _sort16(hk, hv, False)
    lk, lv = _sort16(lk, lv, False)
    return lk, lv, hk, hv


def _merge16kv(ak, av, bk, bv, out_desc=True):
    m = ak >= bk
    hk = jnp.where(m, ak, bk)
    hv = jnp.where(m, av, bv)
    lk = jnp.where(m, bk, ak)
    lv = jnp.where(m, bv, av)
    return _out32(hk, hv, lk, lv, out_desc)


def _merge32kv(a, b, out_desc=True):
    a0k, a0v, a1k, a1v = a
    b0k, b0v, b1k, b1v = b
    m0 = a0k >= b0k
    l0k = jnp.where(m0, a0k, b0k)
    l0v = jnp.where(m0, a0v, b0v)
    m1 = a1k >= b1k
    l1k = jnp.where(m1, a1k, b1k)
    l1v = jnp.where(m1, a1v, b1v)
    m2 = l0k >= l1k
    hk = jnp.where(m2, l0k, l1k)
    hv = jnp.where(m2, l0v, l1v)
    lk = jnp.where(m2, l1k, l0k)
    lv = jnp.where(m2, l1v, l0v)
    return _out32(hk, hv, lk, lv, out_desc)


def _top32_net(pairs):
    units = [_merge16kv(*pairs[2 * j], *pairs[2 * j + 1],
                        out_desc=(j % 2 == 0))
             for j in range(len(pairs) // 2)]
    while len(units) > 1:
        units = [_merge32kv(units[2 * j], units[2 * j + 1],
                            out_desc=(j % 2 == 0))
                 for j in range(len(units) // 2)]
    return units[0]


SCH = 16
NCH = RPW // SCH


def _topk_sc(s2, m1):
    mesh = plsc.VectorSubcoreMesh(core_axis_name="c", subcore_axis_name="s")

    @functools.partial(
        pl.kernel,
        mesh=mesh,
        out_type=jax.ShapeDtypeStruct((ROWS * KP,), jnp.int32),
        compiler_params=pltpu.CompilerParams(needs_layout_passes=False),
        interpret=_INTERPRET,
        scratch_types=[
            pltpu.VMEM((2, SCH, N), jnp.float32),
            pltpu.VMEM((2, SCH, G), jnp.float32),
            pltpu.VMEM((SCH * KP,), jnp.int32),
            pltpu.VMEM((4, KP), jnp.int32),
            pltpu.SemaphoreType.DMA,
            pltpu.SemaphoreType.DMA,
            pltpu.SemaphoreType.DMA,
            pltpu.SemaphoreType.DMA,
        ],
    )
    def topk_kernel(s_hbm, m_hbm, o_hbm, s_buf, m_buf, out_buf, idxv,
                    ss0, ss1, ms0, ms1):
        wid = lax.axis_index("s") * 2 + lax.axis_index("c")
        wbase = wid * RPW
        iot = lax.broadcasted_iota(jnp.int32, (GS,), 0)
        ssem = (ss0, ss1)
        msem = (ms0, ms1)

        def start(c, par):
            rb = pl.multiple_of(wbase + c * SCH, SCH)
            pltpu.make_async_copy(
                s_hbm.at[pl.ds(rb, SCH)], s_buf.at[par], ssem[par]).start()
            pltpu.make_async_copy(
                m_hbm.at[pl.ds(rb, SCH)], m_buf.at[par], msem[par]).start()

        def wait(par):
            pltpu.make_async_copy(
                s_hbm.at[pl.ds(0, SCH)], s_buf.at[par], ssem[par]).wait()
            pltpu.make_async_copy(
                m_hbm.at[pl.ds(0, SCH)], m_buf.at[par], msem[par]).wait()

        def compute(c, par):
            rbase = wbase + c * SCH

            def row_body(r, q, carry_r):
                rabs = rbase + r
                rsplat = jnp.full((GS,), r, jnp.int32)
                leaves = []
                for j in range(G // GS):
                    kj = plsc.load_gather(
                        m_buf.at[par], [rsplat, iot + (GS * j)])
                    leaves.append(_sort16(kj, iot + (GS * j), j % 2 == 0))
                _, ghv, _, glv = _top32_net(leaves)
                idxq = idxv.at[q]
                idxq[pl.ds(0, GS)] = ghv
                idxq[pl.ds(GS, GS)] = glv
                el = []
                for j in range(KP):
                    g = plsc.load_gather(idxq, [jnp.full((GS,), j, jnp.int32)])
                    col = g * GS + iot
                    kj = plsc.load_gather(s_buf.at[par], [rsplat, col])
                    el.append(_sort16(kj, col, j % 2 == 0))
                _, hv, _, lv = _top32_net(el)
                oo = r * KP
                out_buf[pl.ds(oo, GS)] = hv
                pad = jnp.full((GS,), lax.rem(rabs, N), jnp.int32)
                lv = jnp.where(iot >= GS - 2, pad, lv)
                out_buf[pl.ds(oo + GS, GS)] = lv
                return carry_r

            def row_quad(i, carry_r):
                for q in range(4):
                    row_body(4 * i + q, q, carry_r)
                return carry_r

            lax.fori_loop(0, SCH // 4, row_quad, 0)
            pltpu.sync_copy(
                out_buf,
                o_hbm.at[pl.ds(pl.multiple_of(rbase * KP, SCH * KP), SCH * KP)])

        start(0, 0)

        def pair_body(t, carry):
            for par in range(2):
                c = 2 * t + par
                wait(par)

                @pl.when(c + 1 < NCH)
                def _():
                    start(c + 1, 1 - par)

                compute(c, par)
            return carry

        lax.fori_loop(0, NCH // 2, pair_body, 0)

    return topk_kernel(s2, m1)


def _gather_sc(v2, idx):
    mesh = plsc.VectorSubcoreMesh(core_axis_name="c", subcore_axis_name="s")
    epw = ROWS * KP // NW

    @functools.partial(
        pl.kernel,
        mesh=mesh,
        out_type=jax.ShapeDtypeStruct((ROWS * KP // 4, 128), jnp.float32),
        compiler_params=pltpu.CompilerParams(needs_layout_passes=False),
        interpret=_INTERPRET,
        scratch_types=[
            pltpu.VMEM((N * VP,), jnp.float32),
            pltpu.VMEM((ECH,), jnp.int32),
            pltpu.VMEM((ECH // 4, 128), jnp.float32),
        ],
    )
    def gather_kernel(v_hbm, i_hbm, o_hbm, vtab, ibuf, obuf):
        wid = lax.axis_index("s") * 2 + lax.axis_index("c")
        ebase = wid * epw
        b = wid // 2
        pltpu.sync_copy(
            v_hbm.at[pl.ds(pl.multiple_of(b * N * VP, N * VP), N * VP)], vtab)
        iot = lax.broadcasted_iota(jnp.int32, (GS,), 0)
        ec0 = (iot % 4) * 32
        iot4 = iot // 4

        def body(t, carry):
            off = ebase + t * ECH
            pltpu.sync_copy(
                i_hbm.at[pl.ds(pl.multiple_of(off, ECH), ECH)], ibuf)

            def edges16(e0, carry2):
                nids16 = ibuf[pl.ds(e0, GS)] * VP
                for ee in range(GS):
                    nid = nids16[ee]
                    rr = jnp.full((GS,), e0 // 4 + ee // 4, jnp.int32)
                    c0 = (ee % 4) * 32
                    lo = plsc.load_gather(vtab, [nid + iot])
                    hi = plsc.load_gather(vtab, [nid + iot + 16])
                    plsc.store_scatter(obuf, [rr, iot + c0], lo)
                    plsc.store_scatter(obuf, [rr, iot + (c0 + 16)], hi)
                return carry2

            def edges32(i, c):
                edges16(i * 2 * GS, c)
                edges16(i * 2 * GS + GS, c)
                return c

            lax.fori_loop(0, ECH // GS // 2, edges32, 0)
            pltpu.sync_copy(
                obuf, o_hbm.at[pl.ds(pl.multiple_of(off // 4, 256), ECH // 4)])
            return carry

        lax.fori_loop(0, epw // ECH, body, 0)

    return gather_kernel(v2, idx)


RP = TN * KP // 4


def _edge_kernel(u_ref, ve_ref, w2_ref, b2_ref, w3_ref, b3_ref, o_ref):
    u = u_ref[0]
    ve = ve_ref[0]
    u4 = jnp.tile(u, (1, 4))
    ub = jnp.broadcast_to(u4[:, None, :], (TN, KP // 4, 128)).reshape(RP, 128)
    h1 = jnp.maximum(ve + ub, 0.0)
    h2 = jnp.dot(h1, w2_ref[...], preferred_element_type=jnp.float32) + b2_ref[...]
    h2 = jnp.maximum(h2, 0.0)
    msg = jnp.dot(h2, w3_ref[...], preferred_element_type=jnp.float32) + b3_ref[...]
    t = jnp.max(msg.reshape(TN, KP // 4, 128), axis=1)
    o_ref[0] = jnp.maximum(
        jnp.maximum(t[:, 0:32], t[:, 32:64]),
        jnp.maximum(t[:, 64:96], t[:, 96:128]))


def _edge_mlp_max(u, ve, w2, b2, w3, b3):
    w2d = jnp.kron(jnp.eye(4, dtype=jnp.float32), w2)
    w3d = jnp.kron(jnp.eye(4, dtype=jnp.float32), w3)
    b2d = jnp.tile(b2, (1, 4))
    b3d = jnp.tile(b3, (1, 4))
    grid = (B, N // TN)
    return pl.pallas_call(
        _edge_kernel,
        grid=grid,
        in_specs=[
            pl.BlockSpec((1, TN, 32), lambda b, t: (b, t, 0)),
            pl.BlockSpec((1, RP, 128), lambda b, t: (b, t, 0)),
            pl.BlockSpec((128, 128), lambda b, t: (0, 0)),
            pl.BlockSpec((1, 128), lambda b, t: (0, 0)),
            pl.BlockSpec((128, 128), lambda b, t: (0, 0)),
            pl.BlockSpec((1, 128), lambda b, t: (0, 0)),
        ],
        out_specs=pl.BlockSpec((1, TN, 32), lambda b, t: (b, t, 0)),
        out_shape=jax.ShapeDtypeStruct((B, N, 32), jnp.float32),
        interpret=_INTERPRET,
    )(u, ve, w2d, b2d, w3d, b3d)


def _pool_kernel(x1_ref, x2_ref, x3_ref, w_ref, b_ref, o_ref):
    t = pl.program_id(1)
    h = jnp.concatenate([x1_ref[0], x2_ref[0], x3_ref[0]], axis=1)
    h = jnp.dot(h, w_ref[...], preferred_element_type=jnp.float32) + b_ref[...]
    m = jnp.max(h, axis=0, keepdims=True)[None]

    @pl.when(t == 0)
    def _():
        o_ref[...] = m

    @pl.when(t != 0)
    def _():
        o_ref[...] = jnp.maximum(o_ref[...], m)


def _pool(x1, x2, x3, w, b):
    grid = (B, N // TN)
    return pl.pallas_call(
        _pool_kernel,
        grid=grid,
        in_specs=[
            pl.BlockSpec((1, TN, 32), lambda b, t: (b, t, 0)),
            pl.BlockSpec((1, TN, 32), lambda b, t: (b, t, 0)),
            pl.BlockSpec((1, TN, 32), lambda b, t: (b, t, 0)),
            pl.BlockSpec((96, 1024), lambda b, t: (0, 0)),
            pl.BlockSpec((1, 1024), lambda b, t: (0, 0)),
        ],
        out_specs=pl.BlockSpec((1, 1, 1024), lambda b, t: (b, 0, 0)),
        out_shape=jax.ShapeDtypeStruct((B, 1, 1024), jnp.float32),
        interpret=_INTERPRET,
    )(x1, x2, x3, w, b).reshape(B, 1024)


def _head_kernel(g_ref, w0, b0, w1, b1, w2, b2, w3, b3, o_ref):
    g = g_ref[...]
    g = jnp.maximum(jnp.dot(g, w0[...], preferred_element_type=jnp.float32) + b0[...], 0.0)
    g = jnp.maximum(jnp.dot(g, w1[...], preferred_element_type=jnp.float32) + b1[...], 0.0)
    g = jnp.maximum(jnp.dot(g, w2[...], preferred_element_type=jnp.float32) + b2[...], 0.0)
    o_ref[...] = jnp.dot(g, w3[...], preferred_element_type=jnp.float32) + b3[...]


def _head(g, ws):
    ins = []
    specs = [pl.BlockSpec(g.shape, lambda: (0, 0))]
    for w, b in ws:
        ins += [w, b]
        specs += [pl.BlockSpec(w.shape, lambda: (0, 0)),
                  pl.BlockSpec(b.shape, lambda: (0, 0))]
    ncls = ws[-1][0].shape[1]
    return pl.pallas_call(
        _head_kernel,
        in_specs=specs,
        out_specs=pl.BlockSpec((B, ncls), lambda: (0, 0)),
        out_shape=jax.ShapeDtypeStruct((B, ncls), jnp.float32),
        interpret=_INTERPRET,
    )(g, *ins)


def _layer(x, layers):
    d = x.shape[-1]
    p1, p2, p3 = layers
    w1t, b1t = _fold_bn(p1['W'], p1['b'], p1['gamma'], p1['beta'])
    a, c = w1t[:d], w1t[d:]
    wu = a - c
    bu = b1t.reshape(1, 32)
    w2t, b2t = _fold_bn(p2['W'], p2['b'], p2['gamma'], p2['beta'])
    s, u, v, m = _scores(x, wu, bu, c)
    idxg = _topk_sc(s.reshape(ROWS, N), m.reshape(ROWS, G))
    vp = jnp.pad(v, ((0, 0), (0, 0), (0, VP - 32)))
    ve = _gather_sc(vp.reshape(B * N * VP), idxg)
    ve = ve.reshape(B, N * KP // 4, 128)
    return _edge_mlp_max(u, ve, w2t, b2t.reshape(1, 32),
                         p3['W'], p3['b'].reshape(1, 32))


def kernel(data, params):
    x = data
    xs = []
    for li in range(3):
        x = _layer(x, params['conv%d' % li])
        xs.append(x)
    g = _pool(xs[0], xs[1], xs[2], params['lin1']['W'],
              params['lin1']['b'].reshape(1, 1024))
    ws = [(p['W'], p['b'].reshape(1, -1)) for p in params['out']]
    return _head(g, ws)


# speedup vs baseline: 32.0175x; 1.1017x over previous
import functools

import jax
import jax.numpy as jnp
from jax import lax
from jax.experimental import pallas as pl
from jax.experimental.pallas import tpu as pltpu
from jax.experimental.pallas import tpu_sc as plsc

_INTERPRET = False

K = 30
KP = 32
B, N = 16, 2048
TN = 256
GS = 16
G = N // GS
ROWS = B * N
NW = 32
RPW = ROWS // NW
CH = 64
ECH = 1024
VP = 33


def _fold_bn(W, b, gamma, beta):
    s = gamma / jnp.sqrt(1.0 + 1e-5)
    return W * s[None, :], b * s + beta


def _scores_kernel(x_ref, xt_ref, wu_ref, bu_ref, wv_ref, e_ref,
                   s_ref, u_ref, v_ref, m_ref):
    xall = x_ref[0]
    xt = xt_ref[0]
    g = lax.dot_general(xt, xall, (((1,), (1,)), ((), ())),
                        preferred_element_type=jnp.float32)
    x2 = jnp.sum(xall * xall, axis=1)
    s = 2.0 * g - x2[None, :]
    s_ref[0] = s
    t = s
    for k in (1, 2, 4, 8):
        pad = jnp.full((TN, k), -jnp.inf, jnp.float32)
        t = jnp.maximum(t, jnp.concatenate([t[:, k:], pad], axis=1))
    m_ref[0] = jnp.dot(t, e_ref[...], preferred_element_type=jnp.float32)
    u_ref[0] = jnp.dot(xt, wu_ref[...], preferred_element_type=jnp.float32) + bu_ref[...]
    v_ref[0] = jnp.dot(xt, wv_ref[...], preferred_element_type=jnp.float32)


def _scores(x, wu, bu, wv):
    d = x.shape[-1]
    grid = (B, N // TN)
    ext = jnp.zeros((N, G), jnp.float32).at[
        16 * jnp.arange(G), jnp.arange(G)].set(1.0)
    return pl.pallas_call(
        _scores_kernel,
        grid=grid,
        in_specs=[
            pl.BlockSpec((1, N, d), lambda b, t: (b, 0, 0)),
            pl.BlockSpec((1, TN, d), lambda b, t: (b, t, 0)),
            pl.BlockSpec((d, 32), lambda b, t: (0, 0)),
            pl.BlockSpec((1, 32), lambda b, t: (0, 0)),
            pl.BlockSpec((d, 32), lambda b, t: (0, 0)),
            pl.BlockSpec((N, G), lambda b, t: (0, 0)),
        ],
        out_specs=[
            pl.BlockSpec((1, TN, N), lambda b, t: (b, t, 0)),
            pl.BlockSpec((1, TN, 32), lambda b, t: (b, t, 0)),
            pl.BlockSpec((1, TN, 32), lambda b, t: (b, t, 0)),
            pl.BlockSpec((1, TN, G), lambda b, t: (b, t, 0)),
        ],
        out_shape=[
            jax.ShapeDtypeStruct((B, N, N), jnp.float32),
            jax.ShapeDtypeStruct((B, N, 32), jnp.float32),
            jax.ShapeDtypeStruct((B, N, 32), jnp.float32),
            jax.ShapeDtypeStruct((B, N, G), jnp.float32),
        ],
        interpret=_INTERPRET,
    )(x, x, wu, bu, wv, ext)


def _sort16(k, v, desc=True):
    return plsc.sort_key_val(k, v, descending=desc)


def _out32(hk, hv, lk, lv, out_desc):
    if out_desc:
        hk, hv = _sort16(hk, hv, True)
        lk, lv = _sort16(lk, lv, True)
        return hk, hv, lk, lv
    hk, hv = _sort16(hk, hv, False)
    lk, lv = _sort16(lk, lv, False)
    return lk, lv, hk, hv


def _merge16kv(ak, av, bk, bv, out_desc=True):
    m = ak >= bk
    hk = jnp.where(m, ak, bk)
    hv = jnp.where(m, av, bv)
    lk = jnp.where(m, bk, ak)
    lv = jnp.where(m, bv, av)
    return _out32(hk, hv, lk, lv, out_desc)


def _merge32kv(a, b, out_desc=True):
    a0k, a0v, a1k, a1v = a
    b0k, b0v, b1k, b1v = b
    m0 = a0k >= b0k
    l0k = jnp.where(m0, a0k, b0k)
    l0v = jnp.where(m0, a0v, b0v)
    m1 = a1k >= b1k
    l1k = jnp.where(m1, a1k, b1k)
    l1v = jnp.where(m1, a1v, b1v)
    m2 = l0k >= l1k
    hk = jnp.where(m2, l0k, l1k)
    hv = jnp.where(m2, l0v, l1v)
    lk = jnp.where(m2, l1k, l0k)
    lv = jnp.where(m2, l1v, l0v)
    return _out32(hk, hv, lk, lv, out_desc)


def _top32_net(pairs):
    units = [_merge16kv(*pairs[2 * j], *pairs[2 * j + 1],
                        out_desc=(j % 2 == 0))
             for j in range(len(pairs) // 2)]
    while len(units) > 1:
        units = [_merge32kv(units[2 * j], units[2 * j + 1],
                            out_desc=(j % 2 == 0))
                 for j in range(len(units) // 2)]
    return units[0]


SCH = 16
NCH = RPW // SCH


def _topk_sc(s2, m1):
    mesh = plsc.VectorSubcoreMesh(core_axis_name="c", subcore_axis_name="s")

    @functools.partial(
        pl.kernel,
        mesh=mesh,
        out_type=jax.ShapeDtypeStruct((ROWS * KP,), jnp.int32),
        compiler_params=pltpu.CompilerParams(needs_layout_passes=False),
        interpret=_INTERPRET,
        scratch_types=[
            pltpu.VMEM((2, SCH, N), jnp.float32),
            pltpu.VMEM((2, SCH, G), jnp.float32),
            pltpu.VMEM((SCH * KP,), jnp.int32),
            pltpu.SemaphoreType.DMA,
            pltpu.SemaphoreType.DMA,
            pltpu.SemaphoreType.DMA,
            pltpu.SemaphoreType.DMA,
        ],
    )
    def topk_kernel(s_hbm, m_hbm, o_hbm, s_buf, m_buf, out_buf,
                    ss0, ss1, ms0, ms1):
        wid = lax.axis_index("s") * 2 + lax.axis_index("c")
        wbase = wid * RPW
        iot = lax.broadcasted_iota(jnp.int32, (GS,), 0)
        ssem = (ss0, ss1)
        msem = (ms0, ms1)

        def start(c, par):
            rb = pl.multiple_of(wbase + c * SCH, SCH)
            pltpu.make_async_copy(
                s_hbm.at[pl.ds(rb, SCH)], s_buf.at[par], ssem[par]).start()
            pltpu.make_async_copy(
                m_hbm.at[pl.ds(rb, SCH)], m_buf.at[par], msem[par]).start()

        def wait(par):
            pltpu.make_async_copy(
                s_hbm.at[pl.ds(0, SCH)], s_buf.at[par], ssem[par]).wait()
            pltpu.make_async_copy(
                m_hbm.at[pl.ds(0, SCH)], m_buf.at[par], msem[par]).wait()

        def compute(c, par):
            rbase = wbase + c * SCH

            def row_body(r, q, carry_r):
                rabs = rbase + r
                rsplat = jnp.full((GS,), r, jnp.int32)
                leaves = []
                for j in range(G // GS):
                    kj = plsc.load_gather(
                        m_buf.at[par], [rsplat, iot + (GS * j)])
                    leaves.append(_sort16(kj, iot + (GS * j), j % 2 == 0))
                _, ghv, _, glv = _top32_net(leaves)
                el = []
                for j in range(KP):
                    g = ghv[j] if j < GS else glv[j - GS]
                    col = g * GS + iot
                    kj = plsc.load_gather(s_buf.at[par], [rsplat, col])
                    el.append(_sort16(kj, col, j % 2 == 0))
                _, hv, _, lv = _top32_net(el)
                oo = r * KP
                out_buf[pl.ds(oo, GS)] = hv
                pad = jnp.full((GS,), lax.rem(rabs, N), jnp.int32)
                lv = jnp.where(iot >= GS - 2, pad, lv)
                out_buf[pl.ds(oo + GS, GS)] = lv
                return carry_r

            def row_quad(i, carry_r):
                for q in range(4):
                    row_body(4 * i + q, q, carry_r)
                return carry_r

            lax.fori_loop(0, SCH // 4, row_quad, 0)
            pltpu.sync_copy(
                out_buf,
                o_hbm.at[pl.ds(pl.multiple_of(rbase * KP, SCH * KP), SCH * KP)])

        start(0, 0)

        def pair_body(t, carry):
            for par in range(2):
                c = 2 * t + par
                wait(par)

                @pl.when(c + 1 < NCH)
                def _():
                    start(c + 1, 1 - par)

                compute(c, par)
            return carry

        lax.fori_loop(0, NCH // 2, pair_body, 0)

    return topk_kernel(s2, m1)


def _gather_sc(v2, idx):
    mesh = plsc.VectorSubcoreMesh(core_axis_name="c", subcore_axis_name="s")
    epw = ROWS * KP // NW

    @functools.partial(
        pl.kernel,
        mesh=mesh,
        out_type=jax.ShapeDtypeStruct((ROWS * KP // 4, 128), jnp.float32),
        compiler_params=pltpu.CompilerParams(needs_layout_passes=False),
        interpret=_INTERPRET,
        scratch_types=[
            pltpu.VMEM((N * VP,), jnp.float32),
            pltpu.VMEM((ECH,), jnp.int32),
            pltpu.VMEM((ECH // 4, 128), jnp.float32),
        ],
    )
    def gather_kernel(v_hbm, i_hbm, o_hbm, vtab, ibuf, obuf):
        wid = lax.axis_index("s") * 2 + lax.axis_index("c")
        ebase = wid * epw
        b = wid // 2
        pltpu.sync_copy(
            v_hbm.at[pl.ds(pl.multiple_of(b * N * VP, N * VP), N * VP)], vtab)
        iot = lax.broadcasted_iota(jnp.int32, (GS,), 0)
        ec0 = (iot % 4) * 32
        iot4 = iot // 4

        def body(t, carry):
            off = ebase + t * ECH
            pltpu.sync_copy(
                i_hbm.at[pl.ds(pl.multiple_of(off, ECH), ECH)], ibuf)

            def edges16(e0, carry2):
                nids16 = ibuf[pl.ds(e0, GS)] * VP
                for ee in range(GS):
                    nid = nids16[ee]
                    rr = jnp.full((GS,), e0 // 4 + ee // 4, jnp.int32)
                    c0 = (ee % 4) * 32
                    lo = plsc.load_gather(vtab, [nid + iot])
                    hi = plsc.load_gather(vtab, [nid + iot + 16])
                    plsc.store_scatter(obuf, [rr, iot + c0], lo)
                    plsc.store_scatter(obuf, [rr, iot + (c0 + 16)], hi)
                return carry2

            def edges32(i, c):
                edges16(i * 2 * GS, c)
                edges16(i * 2 * GS + GS, c)
                return c

            lax.fori_loop(0, ECH // GS // 2, edges32, 0)
            pltpu.sync_copy(
                obuf, o_hbm.at[pl.ds(pl.multiple_of(off // 4, 256), ECH // 4)])
            return carry

        lax.fori_loop(0, epw // ECH, body, 0)

    return gather_kernel(v2, idx)


RP = TN * KP // 4


def _edge_kernel(u_ref, ve_ref, w2_ref, b2_ref, w3_ref, b3_ref, o_ref):
    u = u_ref[0]
    ve = ve_ref[0]
    u4 = jnp.tile(u, (1, 4))
    ub = jnp.broadcast_to(u4[:, None, :], (TN, KP // 4, 128)).reshape(RP, 128)
    h1 = jnp.maximum(ve + ub, 0.0)
    h2 = jnp.dot(h1, w2_ref[...], preferred_element_type=jnp.float32) + b2_ref[...]
    h2 = jnp.maximum(h2, 0.0)
    msg = jnp.dot(h2, w3_ref[...], preferred_element_type=jnp.float32) + b3_ref[...]
    t = jnp.max(msg.reshape(TN, KP // 4, 128), axis=1)
    o_ref[0] = jnp.maximum(
        jnp.maximum(t[:, 0:32], t[:, 32:64]),
        jnp.maximum(t[:, 64:96], t[:, 96:128]))


def _edge_mlp_max(u, ve, w2, b2, w3, b3):
    w2d = jnp.kron(jnp.eye(4, dtype=jnp.float32), w2)
    w3d = jnp.kron(jnp.eye(4, dtype=jnp.float32), w3)
    b2d = jnp.tile(b2, (1, 4))
    b3d = jnp.tile(b3, (1, 4))
    grid = (B, N // TN)
    return pl.pallas_call(
        _edge_kernel,
        grid=grid,
        in_specs=[
            pl.BlockSpec((1, TN, 32), lambda b, t: (b, t, 0)),
            pl.BlockSpec((1, RP, 128), lambda b, t: (b, t, 0)),
            pl.BlockSpec((128, 128), lambda b, t: (0, 0)),
            pl.BlockSpec((1, 128), lambda b, t: (0, 0)),
            pl.BlockSpec((128, 128), lambda b, t: (0, 0)),
            pl.BlockSpec((1, 128), lambda b, t: (0, 0)),
        ],
        out_specs=pl.BlockSpec((1, TN, 32), lambda b, t: (b, t, 0)),
        out_shape=jax.ShapeDtypeStruct((B, N, 32), jnp.float32),
        interpret=_INTERPRET,
    )(u, ve, w2d, b2d, w3d, b3d)


def _pool_kernel(x1_ref, x2_ref, x3_ref, w_ref, b_ref, o_ref):
    t = pl.program_id(1)
    h = jnp.concatenate([x1_ref[0], x2_ref[0], x3_ref[0]], axis=1)
    h = jnp.dot(h, w_ref[...], preferred_element_type=jnp.float32) + b_ref[...]
    m = jnp.max(h, axis=0, keepdims=True)[None]

    @pl.when(t == 0)
    def _():
        o_ref[...] = m

    @pl.when(t != 0)
    def _():
        o_ref[...] = jnp.maximum(o_ref[...], m)


def _pool(x1, x2, x3, w, b):
    grid = (B, N // TN)
    return pl.pallas_call(
        _pool_kernel,
        grid=grid,
        in_specs=[
            pl.BlockSpec((1, TN, 32), lambda b, t: (b, t, 0)),
            pl.BlockSpec((1, TN, 32), lambda b, t: (b, t, 0)),
            pl.BlockSpec((1, TN, 32), lambda b, t: (b, t, 0)),
            pl.BlockSpec((96, 1024), lambda b, t: (0, 0)),
            pl.BlockSpec((1, 1024), lambda b, t: (0, 0)),
        ],
        out_specs=pl.BlockSpec((1, 1, 1024), lambda b, t: (b, 0, 0)),
        out_shape=jax.ShapeDtypeStruct((B, 1, 1024), jnp.float32),
        interpret=_INTERPRET,
    )(x1, x2, x3, w, b).reshape(B, 1024)


def _head_kernel(g_ref, w0, b0, w1, b1, w2, b2, w3, b3, o_ref):
    g = g_ref[...]
    g = jnp.maximum(jnp.dot(g, w0[...], preferred_element_type=jnp.float32) + b0[...], 0.0)
    g = jnp.maximum(jnp.dot(g, w1[...], preferred_element_type=jnp.float32) + b1[...], 0.0)
    g = jnp.maximum(jnp.dot(g, w2[...], preferred_element_type=jnp.float32) + b2[...], 0.0)
    o_ref[...] = jnp.dot(g, w3[...], preferred_element_type=jnp.float32) + b3[...]


def _head(g, ws):
    ins = []
    specs = [pl.BlockSpec(g.shape, lambda: (0, 0))]
    for w, b in ws:
        ins += [w, b]
        specs += [pl.BlockSpec(w.shape, lambda: (0, 0)),
                  pl.BlockSpec(b.shape, lambda: (0, 0))]
    ncls = ws[-1][0].shape[1]
    return pl.pallas_call(
        _head_kernel,
        in_specs=specs,
        out_specs=pl.BlockSpec((B, ncls), lambda: (0, 0)),
        out_shape=jax.ShapeDtypeStruct((B, ncls), jnp.float32),
        interpret=_INTERPRET,
    )(g, *ins)


def _layer(x, layers):
    d = x.shape[-1]
    p1, p2, p3 = layers
    w1t, b1t = _fold_bn(p1['W'], p1['b'], p1['gamma'], p1['beta'])
    a, c = w1t[:d], w1t[d:]
    wu = a - c
    bu = b1t.reshape(1, 32)
    w2t, b2t = _fold_bn(p2['W'], p2['b'], p2['gamma'], p2['beta'])
    s, u, v, m = _scores(x, wu, bu, c)
    idxg = _topk_sc(s.reshape(ROWS, N), m.reshape(ROWS, G))
    vp = jnp.pad(v, ((0, 0), (0, 0), (0, VP - 32)))
    ve = _gather_sc(vp.reshape(B * N * VP), idxg)
    ve = ve.reshape(B, N * KP // 4, 128)
    return _edge_mlp_max(u, ve, w2t, b2t.reshape(1, 32),
                         p3['W'], p3['b'].reshape(1, 32))


def kernel(data, params):
    x = data
    xs = []
    for li in range(3):
        x = _layer(x, params['conv%d' % li])
        xs.append(x)
    g = _pool(xs[0], xs[1], xs[2], params['lin1']['W'],
              params['lin1']['b'].reshape(1, 1024))
    ws = [(p['W'], p['b'].reshape(1, -1)) for p in params['out']]
    return _head(g, ws)


# speedup vs baseline: 32.0693x; 1.0016x over previous
import functools

import jax
import jax.numpy as jnp
from jax import lax
from jax.experimental import pallas as pl
from jax.experimental.pallas import tpu as pltpu
from jax.experimental.pallas import tpu_sc as plsc

_INTERPRET = False

K = 30
KP = 32
B, N = 16, 2048
TN = 256
GS = 16
G = N // GS
ROWS = B * N
NW = 32
RPW = ROWS // NW
ECH = 1024
VP = 33


def _fold_bn(W, b, gamma, beta):
    s = gamma / jnp.sqrt(1.0 + 1e-5)
    return W * s[None, :], b * s + beta


def _scores_kernel(x_ref, xt_ref, wu_ref, bu_ref, wv_ref, e_ref,
                   s_ref, u_ref, v_ref, m_ref):
    xall = x_ref[0]
    xt = xt_ref[0]
    g = lax.dot_general(xt, xall, (((1,), (1,)), ((), ())),
                        preferred_element_type=jnp.float32)
    x2 = jnp.sum(xall * xall, axis=1)
    s = 2.0 * g - x2[None, :]
    s_ref[0] = s
    t = s
    for k in (1, 2, 4, 8):
        pad = jnp.full((TN, k), -jnp.inf, jnp.float32)
        t = jnp.maximum(t, jnp.concatenate([t[:, k:], pad], axis=1))
    m_ref[0] = jnp.dot(t, e_ref[...], preferred_element_type=jnp.float32)
    u_ref[0] = jnp.dot(xt, wu_ref[...], preferred_element_type=jnp.float32) + bu_ref[...]
    v_ref[0] = jnp.dot(xt, wv_ref[...], preferred_element_type=jnp.float32)


def _scores(x, wu, bu, wv):
    d = x.shape[-1]
    grid = (B, N // TN)
    ext = jnp.zeros((N, G), jnp.float32).at[
        16 * jnp.arange(G), jnp.arange(G)].set(1.0)
    return pl.pallas_call(
        _scores_kernel,
        grid=grid,
        in_specs=[
            pl.BlockSpec((1, N, d), lambda b, t: (b, 0, 0)),
            pl.BlockSpec((1, TN, d), lambda b, t: (b, t, 0)),
            pl.BlockSpec((d, 32), lambda b, t: (0, 0)),
            pl.BlockSpec((1, 32), lambda b, t: (0, 0)),
            pl.BlockSpec((d, 32), lambda b, t: (0, 0)),
            pl.BlockSpec((N, G), lambda b, t: (0, 0)),
        ],
        out_specs=[
            pl.BlockSpec((1, TN, N), lambda b, t: (b, t, 0)),
            pl.BlockSpec((1, TN, 32), lambda b, t: (b, t, 0)),
            pl.BlockSpec((1, TN, 32), lambda b, t: (b, t, 0)),
            pl.BlockSpec((1, TN, G), lambda b, t: (b, t, 0)),
        ],
        out_shape=[
            jax.ShapeDtypeStruct((B, N, N), jnp.float32),
            jax.ShapeDtypeStruct((B, N, 32), jnp.float32),
            jax.ShapeDtypeStruct((B, N, 32), jnp.float32),
            jax.ShapeDtypeStruct((B, N, G), jnp.float32),
        ],
        interpret=_INTERPRET,
    )(x, x, wu, bu, wv, ext)


def _sort16(k, v, desc=True):
    return plsc.sort_key_val(k, v, descending=desc)


def _out32(hk, hv, lk, lv, out_desc):
    if out_desc:
        hk, hv = _sort16(hk, hv, True)
        lk, lv = _sort16(lk, lv, True)
        return hk, hv, lk, lv
    hk, hv = _sort16(hk, hv, False)
    lk, lv = _sort16(lk, lv, False)
    return lk, lv, hk, hv


def _merge16kv(ak, av, bk, bv, out_desc=True):
    m = ak >= bk
    hk = jnp.where(m, ak, bk)
    hv = jnp.where(m, av, bv)
    lk = jnp.where(m, bk, ak)
    lv = jnp.where(m, bv, av)
    return _out32(hk, hv, lk, lv, out_desc)


def _merge32kv(a, b, out_desc=True):
    a0k, a0v, a1k, a1v = a
    b0k, b0v, b1k, b1v = b
    m0 = a0k >= b0k
    l0k = jnp.where(m0, a0k, b0k)
    l0v = jnp.where(m0, a0v, b0v)
    m1 = a1k >= b1k
    l1k = jnp.where(m1, a1k, b1k)
    l1v = jnp.where(m1, a1v, b1v)
    m2 = l0k >= l1k
    hk = jnp.where(m2, l0k, l1k)
    hv = jnp.where(m2, l0v, l1v)
    lk = jnp.where(m2, l1k, l0k)
    lv = jnp.where(m2, l1v, l0v)
    return _out32(hk, hv, lk, lv, out_desc)


def _top32_net(pairs):
    units = [_merge16kv(*pairs[2 * j], *pairs[2 * j + 1],
                        out_desc=(j % 2 == 0))
             for j in range(len(pairs) // 2)]
    while len(units) > 1:
        units = [_merge32kv(units[2 * j], units[2 * j + 1],
                            out_desc=(j % 2 == 0))
                 for j in range(len(units) // 2)]
    return units[0]


SCH = 16
NCH = RPW // SCH


def _topk_sc(s2, m1):
    mesh = plsc.VectorSubcoreMesh(core_axis_name="c", subcore_axis_name="s")

    @functools.partial(
        pl.kernel,
        mesh=mesh,
        out_type=jax.ShapeDtypeStruct((ROWS * KP,), jnp.int32),
        compiler_params=pltpu.CompilerParams(needs_layout_passes=False),
        interpret=_INTERPRET,
        scratch_types=[
            pltpu.VMEM((2, SCH, N), jnp.float32),
            pltpu.VMEM((2, SCH, G), jnp.float32),
            pltpu.VMEM((SCH * KP,), jnp.int32),
            pltpu.SemaphoreType.DMA,
            pltpu.SemaphoreType.DMA,
            pltpu.SemaphoreType.DMA,
            pltpu.SemaphoreType.DMA,
        ],
    )
    def topk_kernel(s_hbm, m_hbm, o_hbm, s_buf, m_buf, out_buf,
                    ss0, ss1, ms0, ms1):
        wid = lax.axis_index("s") * 2 + lax.axis_index("c")
        wbase = wid * RPW
        iot = lax.broadcasted_iota(jnp.int32, (GS,), 0)
        ssem = (ss0, ss1)
        msem = (ms0, ms1)

        def start(c, par):
            rb = pl.multiple_of(wbase + c * SCH, SCH)
            pltpu.make_async_copy(
                s_hbm.at[pl.ds(rb, SCH)], s_buf.at[par], ssem[par]).start()
            pltpu.make_async_copy(
                m_hbm.at[pl.ds(rb, SCH)], m_buf.at[par], msem[par]).start()

        def wait(par):
            pltpu.make_async_copy(
                s_hbm.at[pl.ds(0, SCH)], s_buf.at[par], ssem[par]).wait()
            pltpu.make_async_copy(
                m_hbm.at[pl.ds(0, SCH)], m_buf.at[par], msem[par]).wait()

        def compute(c, par):
            rbase = wbase + c * SCH

            def row_body(r, q, carry_r):
                rabs = rbase + r
                rsplat = jnp.full((GS,), r, jnp.int32)
                leaves = []
                for j in range(G // GS):
                    kj = plsc.load_gather(
                        m_buf.at[par], [rsplat, iot + (GS * j)])
                    leaves.append(_sort16(kj, iot + (GS * j), j % 2 == 0))
                _, ghv, _, glv = _top32_net(leaves)
                el = []
                for j in range(KP):
                    g = ghv[j] if j < GS else glv[j - GS]
                    col = g * GS + iot
                    kj = plsc.load_gather(s_buf.at[par], [rsplat, col])
                    el.append(_sort16(kj, col, j % 2 == 0))
                _, hv, _, lv = _top32_net(el)
                oo = r * KP
                out_buf[pl.ds(oo, GS)] = hv
                pad = jnp.full((GS,), lax.rem(rabs, N), jnp.int32)
                lv = jnp.where(iot >= GS - 2, pad, lv)
                out_buf[pl.ds(oo + GS, GS)] = lv
                return carry_r

            def row_quad(i, carry_r):
                for q in range(4):
                    row_body(4 * i + q, q, carry_r)
                return carry_r

            lax.fori_loop(0, SCH // 4, row_quad, 0)
            pltpu.sync_copy(
                out_buf,
                o_hbm.at[pl.ds(pl.multiple_of(rbase * KP, SCH * KP), SCH * KP)])

        start(0, 0)

        def pair_body(t, carry):
            for par in range(2):
                c = 2 * t + par
                wait(par)

                @pl.when(c + 1 < NCH)
                def _():
                    start(c + 1, 1 - par)

                compute(c, par)
            return carry

        lax.fori_loop(0, NCH // 2, pair_body, 0)

    return topk_kernel(s2, m1)


def _gather_sc(v2, idx):
    mesh = plsc.VectorSubcoreMesh(core_axis_name="c", subcore_axis_name="s")
    epw = ROWS * KP // NW

    @functools.partial(
        pl.kernel,
        mesh=mesh,
        out_type=jax.ShapeDtypeStruct((ROWS * KP // 4, 128), jnp.float32),
        compiler_params=pltpu.CompilerParams(needs_layout_passes=False),
        interpret=_INTERPRET,
        scratch_types=[
            pltpu.VMEM((N * VP,), jnp.float32),
            pltpu.VMEM((ECH,), jnp.int32),
            pltpu.VMEM((ECH // 4, 128), jnp.float32),
        ],
    )
    def gather_kernel(v_hbm, i_hbm, o_hbm, vtab, ibuf, obuf):
        wid = lax.axis_index("s") * 2 + lax.axis_index("c")
        ebase = wid * epw
        b = wid // 2
        pltpu.sync_copy(
            v_hbm.at[pl.ds(pl.multiple_of(b * N * VP, N * VP), N * VP)], vtab)
        iot = lax.broadcasted_iota(jnp.int32, (GS,), 0)
        ec0 = (iot % 4) * 32
        iot4 = iot // 4

        def body(t, carry):
            off = ebase + t * ECH
            pltpu.sync_copy(
                i_hbm.at[pl.ds(pl.multiple_of(off, ECH), ECH)], ibuf)

            def edges16(e0, carry2):
                nids16 = ibuf[pl.ds(e0, GS)] * VP
                for ee in range(GS):
                    nid = nids16[ee]
                    rr = jnp.full((GS,), e0 // 4 + ee // 4, jnp.int32)
                    c0 = (ee % 4) * 32
                    lo = plsc.load_gather(vtab, [nid + iot])
                    hi = plsc.load_gather(vtab, [nid + iot + 16])
                    plsc.store_scatter(obuf, [rr, iot + c0], lo)
                    plsc.store_scatter(obuf, [rr, iot + (c0 + 16)], hi)
                return carry2

            def edges32(i, c):
                edges16(i * 2 * GS, c)
                edges16(i * 2 * GS + GS, c)
                return c

            lax.fori_loop(0, ECH // GS // 2, edges32, 0)
            pltpu.sync_copy(
                obuf, o_hbm.at[pl.ds(pl.multiple_of(off // 4, 256), ECH // 4)])
            return carry

        lax.fori_loop(0, epw // ECH, body, 0)

    return gather_kernel(v2, idx)


RP = TN * KP // 4


def _edge_kernel(u_ref, ve_ref, w2_ref, b2_ref, w3_ref, b3_ref, o_ref):
    u = u_ref[0]
    ve = ve_ref[0]
    u4 = jnp.tile(u, (1, 4))
    ub = jnp.broadcast_to(u4[:, None, :], (TN, KP // 4, 128)).reshape(RP, 128)
    h1 = jnp.maximum(ve + ub, 0.0)
    h2 = jnp.dot(h1, w2_ref[...], preferred_element_type=jnp.float32) + b2_ref[...]
    h2 = jnp.maximum(h2, 0.0)
    msg = jnp.dot(h2, w3_ref[...], preferred_element_type=jnp.float32) + b3_ref[...]
    t = jnp.max(msg.reshape(TN, KP // 4, 128), axis=1)
    o_ref[0] = jnp.maximum(
        jnp.maximum(t[:, 0:32], t[:, 32:64]),
        jnp.maximum(t[:, 64:96], t[:, 96:128]))


def _edge_mlp_max(u, ve, w2, b2, w3, b3):
    w2d = jnp.kron(jnp.eye(4, dtype=jnp.float32), w2)
    w3d = jnp.kron(jnp.eye(4, dtype=jnp.float32), w3)
    b2d = jnp.tile(b2, (1, 4))
    b3d = jnp.tile(b3, (1, 4))
    grid = (B, N // TN)
    return pl.pallas_call(
        _edge_kernel,
        grid=grid,
        in_specs=[
            pl.BlockSpec((1, TN, 32), lambda b, t: (b, t, 0)),
            pl.BlockSpec((1, RP, 128), lambda b, t: (b, t, 0)),
            pl.BlockSpec((128, 128), lambda b, t: (0, 0)),
            pl.BlockSpec((1, 128), lambda b, t: (0, 0)),
            pl.BlockSpec((128, 128), lambda b, t: (0, 0)),
            pl.BlockSpec((1, 128), lambda b, t: (0, 0)),
        ],
        out_specs=pl.BlockSpec((1, TN, 32), lambda b, t: (b, t, 0)),
        out_shape=jax.ShapeDtypeStruct((B, N, 32), jnp.float32),
        interpret=_INTERPRET,
    )(u, ve, w2d, b2d, w3d, b3d)


def _pool_kernel(x1_ref, x2_ref, x3_ref, w_ref, b_ref, o_ref):
    t = pl.program_id(1)
    h = jnp.concatenate([x1_ref[0], x2_ref[0], x3_ref[0]], axis=1)
    h = jnp.dot(h, w_ref[...], preferred_element_type=jnp.float32) + b_ref[...]
    m = jnp.max(h, axis=0, keepdims=True)[None]

    @pl.when(t == 0)
    def _():
        o_ref[...] = m

    @pl.when(t != 0)
    def _():
        o_ref[...] = jnp.maximum(o_ref[...], m)


def _pool(x1, x2, x3, w, b):
    grid = (B, N // TN)
    return pl.pallas_call(
        _pool_kernel,
        grid=grid,
        in_specs=[
            pl.BlockSpec((1, TN, 32), lambda b, t: (b, t, 0)),
            pl.BlockSpec((1, TN, 32), lambda b, t: (b, t, 0)),
            pl.BlockSpec((1, TN, 32), lambda b, t: (b, t, 0)),
            pl.BlockSpec((96, 1024), lambda b, t: (0, 0)),
            pl.BlockSpec((1, 1024), lambda b, t: (0, 0)),
        ],
        out_specs=pl.BlockSpec((1, 1, 1024), lambda b, t: (b, 0, 0)),
        out_shape=jax.ShapeDtypeStruct((B, 1, 1024), jnp.float32),
        interpret=_INTERPRET,
    )(x1, x2, x3, w, b).reshape(B, 1024)


def _head_kernel(g_ref, w0, b0, w1, b1, w2, b2, w3, b3, o_ref):
    g = g_ref[...]
    g = jnp.maximum(jnp.dot(g, w0[...], preferred_element_type=jnp.float32) + b0[...], 0.0)
    g = jnp.maximum(jnp.dot(g, w1[...], preferred_element_type=jnp.float32) + b1[...], 0.0)
    g = jnp.maximum(jnp.dot(g, w2[...], preferred_element_type=jnp.float32) + b2[...], 0.0)
    o_ref[...] = jnp.dot(g, w3[...], preferred_element_type=jnp.float32) + b3[...]


def _head(g, ws):
    ins = []
    specs = [pl.BlockSpec(g.shape, lambda: (0, 0))]
    for w, b in ws:
        ins += [w, b]
        specs += [pl.BlockSpec(w.shape, lambda: (0, 0)),
                  pl.BlockSpec(b.shape, lambda: (0, 0))]
    ncls = ws[-1][0].shape[1]
    return pl.pallas_call(
        _head_kernel,
        in_specs=specs,
        out_specs=pl.BlockSpec((B, ncls), lambda: (0, 0)),
        out_shape=jax.ShapeDtypeStruct((B, ncls), jnp.float32),
        interpret=_INTERPRET,
    )(g, *ins)


def _layer(x, layers):
    d = x.shape[-1]
    p1, p2, p3 = layers
    w1t, b1t = _fold_bn(p1['W'], p1['b'], p1['gamma'], p1['beta'])
    a, c = w1t[:d], w1t[d:]
    wu = a - c
    bu = b1t.reshape(1, 32)
    w2t, b2t = _fold_bn(p2['W'], p2['b'], p2['gamma'], p2['beta'])
    s, u, v, m = _scores(x, wu, bu, c)
    idxg = _topk_sc(s.reshape(ROWS, N), m.reshape(ROWS, G))
    vp = jnp.pad(v, ((0, 0), (0, 0), (0, VP - 32)))
    ve = _gather_sc(vp.reshape(B * N * VP), idxg)
    ve = ve.reshape(B, N * KP // 4, 128)
    return _edge_mlp_max(u, ve, w2t, b2t.reshape(1, 32),
                         p3['W'], p3['b'].reshape(1, 32))


def kernel(data, params):
    x = data
    xs = []
    for li in range(3):
        x = _layer(x, params['conv%d' % li])
        xs.append(x)
    g = _pool(xs[0], xs[1], xs[2], params['lin1']['W'],
              params['lin1']['b'].reshape(1, 1024))
    ws = [(p['W'], p['b'].reshape(1, -1)) for p in params['out']]
    return _head(g, ws)
